# baseline (speedup 1.0000x reference)
import functools
import math
import jax, jax.numpy as jnp
from jax import lax
from jax.experimental import pallas as pl
from jax.experimental.pallas import tpu as pltpu
from jax.experimental.pallas import tpu_sc as plsc

T1 = 5
N = 10000
B = 1
H = 64
NH = 4
DF = 16
DE = 16
SD = 8
IN_SD = 24
TED = 16

EC = 128
NC, NS = 2, 16
NTILES = NC * NS
E0 = 160000
E = E0 + N
E_PAD = ((E + NTILES * EC - 1) // (NTILES * EC)) * (NTILES * EC)
PT = E_PAD // NTILES
NCHUNK = PT // EC
BN = 1000
BE = 1024
NROWS = N + 240
RT = NROWS // NS
NSUB = 4
RD = RT // NSUB

_INTERPRET = False


def _ln(x, g, b):
    m = jnp.mean(x, axis=-1, keepdims=True)
    v = jnp.var(x, axis=-1, keepdims=True)
    return (x - m) / jnp.sqrt(v + 1e-5) * g + b


def _gelu(x):
    return x * 0.5 * (1.0 + lax.erf(x * (1.0 / math.sqrt(2.0))))


def _sel64():
    r = lax.broadcasted_iota(jnp.int32, (H, H), 0) // DF
    c = lax.broadcasted_iota(jnp.int32, (H, H), 1) // DF
    return (r == c).astype(jnp.float32)


def _b16_64():
    r = lax.broadcasted_iota(jnp.int32, (16, H), 0)
    c = lax.broadcasted_iota(jnp.int32, (16, H), 1) // DF
    return (r == c).astype(jnp.float32)


def _tc1_body(x_ref, temb_ref, inw_ref, inb_ref, ing_ref, inbb_ref,
              step_ref, trw_ref, trb_ref, h_ref, tre_ref):
    temb = temb_ref[...]
    te = jnp.dot(temb, inw_ref[...][IN_SD:, :],
                 preferred_element_type=jnp.float32)
    tre_ref[...] = jnp.dot(temb, trw_ref[...],
                           preferred_element_type=jnp.float32) + trb_ref[...]
    w1 = inw_ref[...][:IN_SD, :]
    step = step_ref[...]
    for i in range(T1):
        hi = jnp.dot(x_ref[i], w1, preferred_element_type=jnp.float32)
        hi = hi + te[i:i + 1, :] + inb_ref[...]
        hi = _ln(hi, ing_ref[...], inbb_ref[...])
        h_ref[i] = hi + step


def _tc1(Xt, temb8, in_w, in_b, inn_g, inn_b, step_row, tr_w, tr_b):
    grid = (N // BN,)
    return pl.pallas_call(
        _tc1_body,
        grid=grid,
        in_specs=[
            pl.BlockSpec((T1, BN, IN_SD), lambda i: (0, i, 0)),
            pl.BlockSpec((8, TED), lambda i: (0, 0)),
            pl.BlockSpec((IN_SD + TED, H), lambda i: (0, 0)),
            pl.BlockSpec((H,), lambda i: (0,)),
            pl.BlockSpec((H,), lambda i: (0,)),
            pl.BlockSpec((H,), lambda i: (0,)),
            pl.BlockSpec((1, H), lambda i: (0, 0)),
            pl.BlockSpec((TED, H), lambda i: (0, 0)),
            pl.BlockSpec((H,), lambda i: (0,)),
        ],
        out_specs=[
            pl.BlockSpec((T1, BN, H), lambda i: (0, i, 0)),
            pl.BlockSpec((8, H), lambda i: (0, 0)),
        ],
        out_shape=[
            jax.ShapeDtypeStruct((T1, N, H), jnp.float32),
            jax.ShapeDtypeStruct((8, H), jnp.float32),
        ],
        interpret=_INTERPRET,
    )(Xt, temb8, in_w, in_b, inn_g, inn_b, step_row, tr_w, tr_b)


def _tc2_body(h_ref, tre_ref,
              wq_ref, bq_ref, wk_ref, bk_ref, wv_ref, bv_ref, ow_ref, ob_ref,
              tng_ref, tnb_ref, w1_ref, b1_ref, w2_ref, b2_ref,
              fng_ref, fnb_ref, Qw_ref, Kw_ref, Vw_ref,
              hbt_ref, Qo_ref, KVo_ref):
    sel = _sel64() * (1.0 / math.sqrt(DF))
    h = h_ref[...]
    hf = h.reshape(T1 * BN, H)
    q = jnp.dot(hf, wq_ref[...], preferred_element_type=jnp.float32) + bq_ref[...]
    k = jnp.dot(hf, wk_ref[...], preferred_element_type=jnp.float32) + bk_ref[...]
    v = jnp.dot(hf, wv_ref[...], preferred_element_type=jnp.float32) + bv_ref[...]
    q3 = q.reshape(T1, BN, H)
    k3 = k.reshape(T1, BN, H)
    v3 = v.reshape(T1, BN, H)
    ao = []
    for i in range(T1):
        s = [jnp.dot(q3[i] * k3[j], sel, preferred_element_type=jnp.float32)
             for j in range(T1)]
        m = s[0]
        for j in range(1, T1):
            m = jnp.maximum(m, s[j])
        p = [jnp.exp(s[j] - m) for j in range(T1)]
        z = p[0]
        for j in range(1, T1):
            z = z + p[j]
        acc = p[0] * v3[0]
        for j in range(1, T1):
            acc = acc + p[j] * v3[j]
        ao.append(acc / z)
    aof = jnp.stack(ao, axis=0).reshape(T1 * BN, H)
    aof = jnp.dot(aof, ow_ref[...], preferred_element_type=jnp.float32) + ob_ref[...]
    h2 = _ln(hf + aof, tng_ref[...], tnb_ref[...])
    ffn = jnp.dot(_gelu(jnp.dot(h2, w1_ref[...], preferred_element_type=jnp.float32) + b1_ref[...]),
                  w2_ref[...], preferred_element_type=jnp.float32) + b2_ref[...]
    h3 = _ln(h2 + ffn, fng_ref[...], fnb_ref[...])
    h3 = h3.reshape(T1, BN, H)
    tre = tre_ref[...]
    h3 = h3 + jnp.concatenate([tre[i:i + 1] for i in range(T1)], axis=0)[:, None, :]
    hbt_ref[...] = h3
    hf2 = h3.reshape(T1 * BN, H)
    Qo_ref[...] = jnp.dot(hf2, Qw_ref[...],
                          preferred_element_type=jnp.float32).reshape(T1, BN, H)
    kk = jnp.dot(hf2, Kw_ref[...], preferred_element_type=jnp.float32).reshape(T1, BN, H)
    vv = jnp.dot(hf2, Vw_ref[...], preferred_element_type=jnp.float32).reshape(T1, BN, H)
    KVo_ref[...] = jnp.concatenate([kk, vv], axis=-1)


def _tc2(h, tre, lp):
    grid = (N // BN,)
    wspec = pl.BlockSpec((H, H), lambda i: (0, 0))
    bspec = pl.BlockSpec((H,), lambda i: (0,))
    hspec = pl.BlockSpec((T1, BN, H), lambda i: (0, i, 0))
    return pl.pallas_call(
        _tc2_body,
        grid=grid,
        in_specs=[
            hspec,
            pl.BlockSpec((8, H), lambda i: (0, 0)),
            wspec, bspec, wspec, bspec, wspec, bspec, wspec, bspec,
            bspec, bspec,
            pl.BlockSpec((H, 2 * H), lambda i: (0, 0)),
            pl.BlockSpec((2 * H,), lambda i: (0,)),
            pl.BlockSpec((2 * H, H), lambda i: (0, 0)),
            bspec,
            bspec, bspec,
            wspec, wspec, wspec,
        ],
        out_specs=[hspec, hspec, pl.BlockSpec((T1, BN, 2 * H), lambda i: (0, i, 0))],
        out_shape=[jax.ShapeDtypeStruct((T1, N, H), jnp.float32),
                   jax.ShapeDtypeStruct((T1, NROWS, H), jnp.float32),
                   jax.ShapeDtypeStruct((T1, NROWS, 2 * H), jnp.float32)],
        interpret=_INTERPRET,
    )(h, tre,
      lp['attn_wq'], lp['attn_bq'], lp['attn_wk'], lp['attn_bk'],
      lp['attn_wv'], lp['attn_bv'], lp['attn_ow'], lp['attn_ob'],
      lp['tn_g'], lp['tn_b'], lp['ffn_w1'], lp['ffn_b1'],
      lp['ffn_w2'], lp['ffn_b2'], lp['fn_g'], lp['fn_b'],
      lp['Q'], lp['K'], lp['V'])


def _tc3_body(hbt_ref, nd_ref, ow_ref, ob_ref, gng_ref, gnb_ref, ew1_ref,
              h_ref, pd_ref, ps_ref, *, first):
    b16 = _b16_64()
    nd = nd_ref[0] + nd_ref[1]
    ndf = nd.reshape(T1 * BN, 80)
    num = ndf[:, :H]
    den = jnp.dot(ndf[:, H:], b16, preferred_element_type=jnp.float32) + 1e-9
    out = num / den
    hbt = hbt_ref[...].reshape(T1 * BN, H)
    hbt = hbt + jnp.dot(out, ow_ref[...], preferred_element_type=jnp.float32) + ob_ref[...]
    if first:
        hm = hbt.reshape(T1, BN, H)
        hmean = (hm[0] + hm[1] + hm[2] + hm[3] + hm[4]) * (1.0 / T1)
        ew1 = ew1_ref[...]
        pd_ref[...] = jnp.dot(hmean, ew1[:H, :], preferred_element_type=jnp.float32)
        ps_ref[...] = jnp.dot(hmean, ew1[H:2 * H, :], preferred_element_type=jnp.float32)
    h_ref[...] = _ln(hbt, gng_ref[...], gnb_ref[...]).reshape(T1, BN, H)


def _tc3(hbt, numden, lp, first):
    grid = (N // BN,)
    hspec = pl.BlockSpec((T1, BN, H), lambda i: (0, i, 0))
    nspec = pl.BlockSpec((2, T1, BN, 80), lambda i: (0, 0, i, 0))
    pspec = pl.BlockSpec((BN, H), lambda i: (i, 0))
    out_specs = [hspec, pspec, pspec]
    out_shape = [jax.ShapeDtypeStruct((T1, N, H), jnp.float32),
                 jax.ShapeDtypeStruct((N, H), jnp.float32),
                 jax.ShapeDtypeStruct((N, H), jnp.float32)]
    res = pl.pallas_call(
        functools.partial(_tc3_body, first=first),
        grid=grid,
        in_specs=[
            hspec, nspec,
            pl.BlockSpec((H, H), lambda i: (0, 0)),
            pl.BlockSpec((H,), lambda i: (0,)),
            pl.BlockSpec((H,), lambda i: (0,)),
            pl.BlockSpec((H,), lambda i: (0,)),
            pl.BlockSpec((2 * H + DE, 4 * DE), lambda i: (0, 0)),
        ],
        out_specs=out_specs,
        out_shape=out_shape,
        interpret=_INTERPRET,
    )(hbt, numden, lp['ow'], lp['ob'], lp['gn_g'], lp['gn_b'], lp['ew1'])
    return res


def _tce0_body(gd_ref, gs_ref, wd_ref, ws_ref, epb_ref, w0_ref, b0_ref,
               e_ref, mod_ref):
    e = (jnp.dot(gd_ref[...], wd_ref[...], preferred_element_type=jnp.float32)
         + jnp.dot(gs_ref[...], ws_ref[...], preferred_element_type=jnp.float32)
         + epb_ref[...])
    e_ref[...] = e
    mod_ref[...] = jnp.dot(e, w0_ref[...], preferred_element_type=jnp.float32) + b0_ref[...]


def _tce0(gdd, gds, Wd, Ws, ep_b, w0, b0):
    grid = (E_PAD // BE,)
    return pl.pallas_call(
        _tce0_body,
        grid=grid,
        in_specs=[
            pl.BlockSpec((BE, 16), lambda i: (i, 0)),
            pl.BlockSpec((BE, 16), lambda i: (i, 0)),
            pl.BlockSpec((16, DE), lambda i: (0, 0)),
            pl.BlockSpec((16, DE), lambda i: (0, 0)),
            pl.BlockSpec((DE,), lambda i: (0,)),
            pl.BlockSpec((DE, 16), lambda i: (0, 0)),
            pl.BlockSpec((16,), lambda i: (0,)),
        ],
        out_specs=[pl.BlockSpec((BE, DE), lambda i: (i, 0)),
                   pl.BlockSpec((BE, 16), lambda i: (i, 0))],
        out_shape=[jax.ShapeDtypeStruct((E_PAD, DE), jnp.float32),
                   jax.ShapeDtypeStruct((E_PAD, 16), jnp.float32)],
        interpret=_INTERPRET,
    )(gdd, gds, Wd, Ws, ep_b, w0, b0)


def _tc4_body(e_ref, gd_ref, gs_ref, ew1_ref, eb1_ref, ew2_ref, eb2_ref,
              eng_ref, enb_ref, w1_ref, b1_ref, e_out_ref, mod_ref):
    e = e_ref[...]
    z = gd_ref[...] + gs_ref[...] + eb1_ref[...] + jnp.dot(
        e, ew1_ref[...][2 * H:, :], preferred_element_type=jnp.float32)
    em = jnp.dot(_gelu(z), ew2_ref[...], preferred_element_type=jnp.float32) + eb2_ref[...]
    e_new = _ln(e + em, eng_ref[...], enb_ref[...])
    e_out_ref[...] = e_new
    mod_ref[...] = jnp.dot(e_new, w1_ref[...], preferred_element_type=jnp.float32) + b1_ref[...]


def _tc4(e, gd, gs, lp, w1mod, b1mod):
    grid = (E_PAD // BE,)
    return pl.pallas_call(
        _tc4_body,
        grid=grid,
        in_specs=[
            pl.BlockSpec((BE, DE), lambda i: (i, 0)),
            pl.BlockSpec((BE, H), lambda i: (i, 0)),
            pl.BlockSpec((BE, H), lambda i: (i, 0)),
            pl.BlockSpec((2 * H + DE, 4 * DE), lambda i: (0, 0)),
            pl.BlockSpec((4 * DE,), lambda i: (0,)),
            pl.BlockSpec((4 * DE, DE), lambda i: (0, 0)),
            pl.BlockSpec((DE,), lambda i: (0,)),
            pl.BlockSpec((DE,), lambda i: (0,)),
            pl.BlockSpec((DE,), lambda i: (0,)),
            pl.BlockSpec((DE, 16), lambda i: (0, 0)),
            pl.BlockSpec((16,), lambda i: (0,)),
        ],
        out_specs=[pl.BlockSpec((BE, DE), lambda i: (i, 0)),
                   pl.BlockSpec((BE, 16), lambda i: (i, 0))],
        out_shape=[jax.ShapeDtypeStruct((E_PAD, DE), jnp.float32),
                   jax.ShapeDtypeStruct((E_PAD, 16), jnp.float32)],
        interpret=_INTERPRET,
    )(e, gd, gs, lp['ew1'], lp['eb1'], lp['ew2'], lp['eb2'],
      lp['en_g'], lp['en_b'], w1mod, b1mod)


def _tc5_body(h_ref, w_ref, b_ref, o_ref):
    z = jnp.dot(h_ref[...], w_ref[...], preferred_element_type=jnp.float32) + b_ref[...]
    z = z - jnp.max(z, axis=-1, keepdims=True)
    ez = jnp.exp(z)
    o_ref[...] = ez / jnp.sum(ez, axis=-1, keepdims=True)


def _tc5(h2d, out_w, out_b):
    R = h2d.shape[0]
    BR = 5000
    return pl.pallas_call(
        _tc5_body,
        grid=(R // BR,),
        in_specs=[
            pl.BlockSpec((BR, H), lambda i: (i, 0)),
            pl.BlockSpec((H, SD), lambda i: (0, 0)),
            pl.BlockSpec((SD,), lambda i: (0,)),
        ],
        out_specs=pl.BlockSpec((BR, SD), lambda i: (i, 0)),
        out_shape=jax.ShapeDtypeStruct((R, SD), jnp.float32),
        interpret=_INTERPRET,
    )(h2d, out_w, out_b)


def _tc_deg_combine_body(dp_ref, o_ref):
    d = dp_ref[0] + dp_ref[1]
    l16 = lax.broadcasted_iota(jnp.int32, (NROWS, 16), 1)
    degb = jnp.where(l16 == 0, d, 0.0)
    deg = jnp.broadcast_to(jnp.sum(degb, axis=-1, keepdims=True), (NROWS, 16))
    rows = lax.broadcasted_iota(jnp.int32, (NROWS, 16), 0)
    lanes = lax.broadcasted_iota(jnp.int32, (NROWS, 16), 1)
    degv = jnp.where((rows < N) & (lanes == 0), deg, 0.0)
    dmax = jnp.maximum(jnp.max(degv), 1.0)
    o = jnp.where(lanes == 0, deg,
                  jnp.where(lanes == 1, deg / dmax,
                            jnp.where(lanes == 2, 1.0 / jnp.maximum(deg, 1.0), 0.0)))
    o_ref[...] = o


def _tc_deg_combine(deg_partial):
    return pl.pallas_call(
        _tc_deg_combine_body,
        grid=(1,),
        in_specs=[pl.BlockSpec((2, NROWS, 16), lambda i: (0, 0, 0))],
        out_specs=pl.BlockSpec((NROWS, 16), lambda i: (0, 0)),
        out_shape=jax.ShapeDtypeStruct((NROWS, 16), jnp.float32),
        interpret=_INTERPRET,
    )(deg_partial)


def _sc_mesh():
    return plsc.VectorSubcoreMesh(core_axis_name="c", subcore_axis_name="s",
                                  num_cores=NC, num_subcores=NS)


def _zero_fill(ref, rows, width):
    z = jnp.zeros((16,), jnp.float32)

    def row(r, _):
        for u in range(width // 16):
            ref[r, pl.ds(u * 16, 16)] = z
        return ()

    lax.fori_loop(0, rows, row, ())


def _sc_deg(dstp):
    @functools.partial(
        pl.kernel,
        out_type=jax.ShapeDtypeStruct((NC, NROWS, 16), jnp.float32),
        mesh=_sc_mesh(),
        compiler_params=pltpu.CompilerParams(needs_layout_passes=False, use_tc_tiling_on_sc=False),
        scratch_types=[
            pltpu.VMEM((EC,), jnp.int32),
            pltpu.VMEM((EC, 16), jnp.float32),
            pltpu.VMEM((RT, 16), jnp.float32),
            pltpu.VMEM_SHARED((NROWS, 16), jnp.float32),
        ],
    )
    def k(dh, out, idxd, onesb, rdtmp, acc):
        cid = lax.axis_index("c")
        sid = lax.axis_index("s")
        wid = sid * NC + cid

        if True:
            ov = jnp.where(lax.iota(jnp.int32, 16) == 0, 1.0, 0.0)

            def orow(r, _):
                onesb[r, pl.ds(0, 16)] = ov
                return ()

            lax.fori_loop(0, EC, orow, ())
            _zero_fill(rdtmp, RT, 16)
            pltpu.sync_copy(rdtmp, acc.at[pl.ds(sid * RT, RT)])
            plsc.subcore_barrier()

            def chunk(j, _):
                base = wid * PT + j * EC
                pltpu.sync_copy(dh.at[pl.ds(base, EC)], idxd)
                pltpu.sync_copy(onesb, acc.at[idxd], add=True)
                return ()

            lax.fori_loop(0, NCHUNK, chunk, ())
            plsc.subcore_barrier()
            pltpu.sync_copy(acc.at[pl.ds(sid * RT, RT)], rdtmp)
            pltpu.sync_copy(rdtmp, out.at[cid, pl.ds(sid * RT, RT)])

    return k(dstp)


def _sc_gather_pair(tabD, tabS, dstp, srcp):
    W = tabD.shape[1]

    @functools.partial(
        pl.kernel,
        out_type=[jax.ShapeDtypeStruct((E_PAD, W), jnp.float32)] * 2,
        mesh=_sc_mesh(),
        compiler_params=pltpu.CompilerParams(needs_layout_passes=False, use_tc_tiling_on_sc=False),
        scratch_types=[
            pltpu.VMEM((EC,), jnp.int32),
            pltpu.VMEM((EC,), jnp.int32),
            pltpu.VMEM((EC, W), jnp.float32),
            pltpu.VMEM((EC, W), jnp.float32),
            pltpu.SemaphoreType.DMA,
        ],
    )
    def k(td, ts, dh, sh, gd, gs, idxd, idxs, bd, bs, sem):
        cid = lax.axis_index("c")
        sid = lax.axis_index("s")
        wid = sid * NC + cid

        def chunk(j, _):
            base = wid * PT + j * EC
            pltpu.sync_copy(dh.at[pl.ds(base, EC)], idxd)
            pltpu.sync_copy(sh.at[pl.ds(base, EC)], idxs)
            c1 = pltpu.async_copy(td.at[idxd], bd, sem)
            c2 = pltpu.async_copy(ts.at[idxs], bs, sem)
            c1.wait()
            c2.wait()
            c3 = pltpu.async_copy(bd, gd.at[pl.ds(base, EC)], sem)
            c4 = pltpu.async_copy(bs, gs.at[pl.ds(base, EC)], sem)
            c3.wait()
            c4.wait()
            return ()

        lax.fori_loop(0, NCHUNK, chunk, ())

    return k(tabD, tabS, dstp, srcp)


def _sc_attn(Qt, KVt, dstp, srcp, mod):
    @functools.partial(
        pl.kernel,
        out_type=jax.ShapeDtypeStruct((NC, T1, NROWS, 80), jnp.float32),
        mesh=_sc_mesh(),
        compiler_params=pltpu.CompilerParams(needs_layout_passes=False, use_tc_tiling_on_sc=False),
        scratch_types=[
            pltpu.VMEM((PT,), jnp.int32),
            pltpu.VMEM((PT,), jnp.int32),
            pltpu.VMEM((EC,), jnp.int32),
            pltpu.VMEM((EC,), jnp.int32),
            pltpu.VMEM((EC,), jnp.int32),
            pltpu.VMEM((EC,), jnp.int32),
            pltpu.VMEM((EC, H), jnp.float32),
            pltpu.VMEM((EC, 2 * H), jnp.float32),
            pltpu.VMEM((EC, 16), jnp.float32),
            pltpu.VMEM((EC, 80), jnp.float32),
            pltpu.VMEM((EC, 80), jnp.float32),
            pltpu.VMEM((RD, 80), jnp.float32),
            pltpu.SemaphoreType.DMA,
            pltpu.SemaphoreType.DMA,
            pltpu.VMEM_SHARED((NROWS, 80), jnp.float32),
        ],
    )
    def k(q_hbm, kv_hbm, dst_hbm, src_hbm, mod_hbm, out_hbm,
          idxall_d, idxall_s, idxd0, idxd1, idxq, idxk, qbuf, kvbuf, modbuf,
          contrib0, contrib1, rdtmp, sem, sem2, acc):
        cid = lax.axis_index("c")
        sid = lax.axis_index("s")
        wid = sid * NC + cid
        pltpu.sync_copy(dst_hbm.at[pl.ds(wid * PT, PT)], idxall_d)
        pltpu.sync_copy(src_hbm.at[pl.ds(wid * PT, PT)], idxall_s)

        if True:
            _zero_fill(contrib0, EC, 80)
            _zero_fill(contrib1, EC, 80)

            def t_body(t, _):
                _zero_fill(rdtmp, RD, 80)
                for u in range(NSUB):
                    pltpu.sync_copy(rdtmp, acc.at[pl.ds(sid * RT + u * RD, RD)])
                plsc.subcore_barrier()
                toff = t * NROWS
                lane = lax.iota(jnp.int32, 16)
                zero16 = jnp.zeros((16,), jnp.float32)

                def fold(v):
                    for st in (8, 4, 2, 1):
                        v = v + v.at[lane ^ st].get(mode='promise_in_bounds')
                    return v

                def do_chunk(j, idxd, contrib):
                    base = wid * PT + j * EC

                    def mkidx(kk, _):
                        sl = pl.ds(kk * 16, 16)
                        dv = idxall_d[pl.ds(j * EC + kk * 16, 16)]
                        sv = idxall_s[pl.ds(j * EC + kk * 16, 16)]
                        idxd[sl] = dv
                        idxq[sl] = dv + toff
                        idxk[sl] = sv + toff
                        return ()

                    lax.fori_loop(0, EC // 16, mkidx, ())
                    c1 = pltpu.async_copy(q_hbm.at[idxq], qbuf, sem)
                    c2 = pltpu.async_copy(kv_hbm.at[idxk], kvbuf, sem)
                    c4 = pltpu.async_copy(mod_hbm.at[pl.ds(base, EC)], modbuf, sem)
                    c1.wait()
                    c2.wait()
                    c4.wait()

                    def edge(ee, _):
                        mrow = modbuf[ee, pl.ds(0, 16)]
                        wv = zero16
                        for h in range(NH):
                            qh = qbuf[ee, pl.ds(h * DF, DF)]
                            kh = kvbuf[ee, pl.ds(h * DF, DF)]
                            dv = fold(qh * kh)
                            sh = dv * jnp.broadcast_to(mrow[h], (16,)) +                                 jnp.broadcast_to(mrow[NH + h], (16,))
                            wh = jnp.exp(sh)
                            contrib[ee, pl.ds(h * DF, DF)] = kvbuf[ee, pl.ds(H + h * DF, DF)] * wh
                            wv = wv + jnp.where(lane == h, wh, zero16)
                        contrib[ee, pl.ds(H, 16)] = jnp.where(lane < NH, wv, zero16)
                        return ()

                    lax.fori_loop(0, EC, edge, ())
                    pltpu.async_copy(contrib, acc.at[idxd], sem2, add=True)
                    return ()

                def chunk2(j2, _):
                    @pl.when(j2 > 0)
                    def _():
                        pltpu.make_async_copy(contrib0, acc.at[idxd0], sem2).wait()
                    do_chunk(j2 * 2, idxd0, contrib0)

                    @pl.when(j2 > 0)
                    def _():
                        pltpu.make_async_copy(contrib1, acc.at[idxd1], sem2).wait()
                    do_chunk(j2 * 2 + 1, idxd1, contrib1)
                    return ()

                lax.fori_loop(0, NCHUNK // 2, chunk2, ())
                pltpu.make_async_copy(contrib0, acc.at[idxd0], sem2).wait()
                pltpu.make_async_copy(contrib1, acc.at[idxd1], sem2).wait()
                plsc.subcore_barrier()
                for u in range(NSUB):
                    pltpu.sync_copy(acc.at[pl.ds(sid * RT + u * RD, RD)], rdtmp)
                    pltpu.sync_copy(rdtmp, out_hbm.at[cid, t, pl.ds(sid * RT + u * RD, RD)])
                plsc.subcore_barrier()
                return ()

            lax.fori_loop(0, T1, t_body, ())

    return k(Qt, KVt, dstp, srcp, mod)


def kernel(A, X_k, k_index, edge_index, params):
    del A
    Bn = X_k.shape[0]
    loop = jnp.arange(N, dtype=edge_index.dtype)
    ei = jnp.concatenate([edge_index, jnp.stack([loop, loop])], axis=1)
    dst = ei[0].astype(jnp.int32)
    src = ei[1].astype(jnp.int32)
    npad = E_PAD - E
    dstp = jnp.concatenate([dst, jnp.full((npad,), N, jnp.int32)])
    srcp = jnp.concatenate([src, jnp.zeros((npad,), jnp.int32)])

    p = params
    temb8 = jnp.concatenate([p['time_embed'], jnp.zeros((3, TED), jnp.float32)], axis=0)
    step_row = p['step_embed'][k_index]
    Xt = X_k.reshape(N, T1, IN_SD).transpose(1, 0, 2)

    h, tre = _tc1(Xt, temb8, p['in_w'], p['in_b'], p['inn_g'], p['inn_b'],
                  step_row, p['tr_w'], p['tr_b'])

    deg_partial = _sc_deg(dstp)
    degC = _tc_deg_combine(deg_partial)
    gdd, gds = _sc_gather_pair(degC, degC, dstp, srcp)

    Wd = jnp.zeros((16, DE), jnp.float32).at[2].set(p['ep_w'][0]).at[1].set(p['ep_w'][2])
    Ws = jnp.zeros((16, DE), jnp.float32).at[1].set(p['ep_w'][1])

    def modwb(lp):
        w = jnp.concatenate([lp['emul_w'] * 0.25, lp['eadd_w'],
                             jnp.zeros((DE, 8), jnp.float32)], axis=1)
        b = jnp.concatenate([(lp['emul_b'] + 1.0) * 0.25, lp['eadd_b'],
                             jnp.zeros((8,), jnp.float32)], axis=0)
        return w, b

    w0, b0 = modwb(p['layers'][0])
    w1m, b1m = modwb(p['layers'][1])
    e, mod = _tce0(gdd, gds, Wd, Ws, p['ep_b'], w0, b0)

    for li, lp in enumerate(p['layers']):
        first = (li == 0)
        hbt, Qo, KVo = _tc2(h, tre, lp)

        ndfull = _sc_attn(Qo.reshape(T1 * NROWS, H), KVo.reshape(T1 * NROWS, 2 * H),
                          dstp, srcp, mod)
        numden = ndfull[:, :, :N, :]

        h, pd, ps = _tc3(hbt, numden, lp, first)

        if first:
            zpad = jnp.zeros((NROWS - N, H), jnp.float32)
            gd, gs = _sc_gather_pair(jnp.concatenate([pd, zpad], axis=0),
                                     jnp.concatenate([ps, zpad], axis=0),
                                     dstp, srcp)
            e, mod = _tc4(e, gd, gs, lp, w1m, b1m)

    o2d = _tc5(h.reshape(T1 * N, H), p['out_w'], p['out_b'])
    return o2d.reshape(T1, N, SD).transpose(1, 0, 2).reshape(Bn, N, T1, SD)


# speedup vs baseline: 1.3612x; 1.3612x over previous
import functools
import math
import jax, jax.numpy as jnp
from jax import lax
from jax.experimental import pallas as pl
from jax.experimental.pallas import tpu as pltpu
from jax.experimental.pallas import tpu_sc as plsc

T1 = 5
N = 10000
B = 1
H = 64
NH = 4
DF = 16
DE = 16
SD = 8
IN_SD = 24
TED = 16

EC = 128
NC, NS = 2, 16
NTILES = NC * NS
E0 = 160000
E = E0 + N
E_PAD = ((E + NTILES * EC - 1) // (NTILES * EC)) * (NTILES * EC)
PT = E_PAD // NTILES
NCHUNK = PT // EC
BN = 1000
BE = 1024
NROWS = N + 240
RT = NROWS // NS
NSUB = 4
RD = RT // NSUB

_INTERPRET = False


def _ln(x, g, b):
    m = jnp.mean(x, axis=-1, keepdims=True)
    v = jnp.var(x, axis=-1, keepdims=True)
    return (x - m) / jnp.sqrt(v + 1e-5) * g + b


def _gelu(x):
    return x * 0.5 * (1.0 + lax.erf(x * (1.0 / math.sqrt(2.0))))


def _sel64():
    r = lax.broadcasted_iota(jnp.int32, (H, H), 0) // DF
    c = lax.broadcasted_iota(jnp.int32, (H, H), 1) // DF
    return (r == c).astype(jnp.float32)


def _b16_64():
    r = lax.broadcasted_iota(jnp.int32, (16, H), 0)
    c = lax.broadcasted_iota(jnp.int32, (16, H), 1) // DF
    return (r == c).astype(jnp.float32)


def _tc1_body(x_ref, temb_ref, inw_ref, inb_ref, ing_ref, inbb_ref,
              step_ref, trw_ref, trb_ref, h_ref, tre_ref):
    temb = temb_ref[...]
    te = jnp.dot(temb, inw_ref[...][IN_SD:, :],
                 preferred_element_type=jnp.float32)
    tre_ref[...] = jnp.dot(temb, trw_ref[...],
                           preferred_element_type=jnp.float32) + trb_ref[...]
    w1 = inw_ref[...][:IN_SD, :]
    step = step_ref[...]
    for i in range(T1):
        hi = jnp.dot(x_ref[i], w1, preferred_element_type=jnp.float32)
        hi = hi + te[i:i + 1, :] + inb_ref[...]
        hi = _ln(hi, ing_ref[...], inbb_ref[...])
        h_ref[i] = hi + step


def _tc1(Xt, temb8, in_w, in_b, inn_g, inn_b, step_row, tr_w, tr_b):
    grid = (N // BN,)
    return pl.pallas_call(
        _tc1_body,
        grid=grid,
        in_specs=[
            pl.BlockSpec((T1, BN, IN_SD), lambda i: (0, i, 0)),
            pl.BlockSpec((8, TED), lambda i: (0, 0)),
            pl.BlockSpec((IN_SD + TED, H), lambda i: (0, 0)),
            pl.BlockSpec((H,), lambda i: (0,)),
            pl.BlockSpec((H,), lambda i: (0,)),
            pl.BlockSpec((H,), lambda i: (0,)),
            pl.BlockSpec((1, H), lambda i: (0, 0)),
            pl.BlockSpec((TED, H), lambda i: (0, 0)),
            pl.BlockSpec((H,), lambda i: (0,)),
        ],
        out_specs=[
            pl.BlockSpec((T1, BN, H), lambda i: (0, i, 0)),
            pl.BlockSpec((8, H), lambda i: (0, 0)),
        ],
        out_shape=[
            jax.ShapeDtypeStruct((T1, N, H), jnp.float32),
            jax.ShapeDtypeStruct((8, H), jnp.float32),
        ],
        interpret=_INTERPRET,
    )(Xt, temb8, in_w, in_b, inn_g, inn_b, step_row, tr_w, tr_b)


def _tc2_body(h_ref, tre_ref,
              wq_ref, bq_ref, wk_ref, bk_ref, wv_ref, bv_ref, ow_ref, ob_ref,
              tng_ref, tnb_ref, w1_ref, b1_ref, w2_ref, b2_ref,
              fng_ref, fnb_ref, Qw_ref, Kw_ref, Vw_ref,
              hbt_ref, Qo_ref, KVo_ref):
    sel = _sel64() * (1.0 / math.sqrt(DF))
    h = h_ref[...]
    hf = h.reshape(T1 * BN, H)
    q = jnp.dot(hf, wq_ref[...], preferred_element_type=jnp.float32) + bq_ref[...]
    k = jnp.dot(hf, wk_ref[...], preferred_element_type=jnp.float32) + bk_ref[...]
    v = jnp.dot(hf, wv_ref[...], preferred_element_type=jnp.float32) + bv_ref[...]
    q3 = q.reshape(T1, BN, H)
    k3 = k.reshape(T1, BN, H)
    v3 = v.reshape(T1, BN, H)
    ao = []
    for i in range(T1):
        s = [jnp.dot(q3[i] * k3[j], sel, preferred_element_type=jnp.float32)
             for j in range(T1)]
        m = s[0]
        for j in range(1, T1):
            m = jnp.maximum(m, s[j])
        p = [jnp.exp(s[j] - m) for j in range(T1)]
        z = p[0]
        for j in range(1, T1):
            z = z + p[j]
        acc = p[0] * v3[0]
        for j in range(1, T1):
            acc = acc + p[j] * v3[j]
        ao.append(acc / z)
    aof = jnp.stack(ao, axis=0).reshape(T1 * BN, H)
    aof = jnp.dot(aof, ow_ref[...], preferred_element_type=jnp.float32) + ob_ref[...]
    h2 = _ln(hf + aof, tng_ref[...], tnb_ref[...])
    ffn = jnp.dot(_gelu(jnp.dot(h2, w1_ref[...], preferred_element_type=jnp.float32) + b1_ref[...]),
                  w2_ref[...], preferred_element_type=jnp.float32) + b2_ref[...]
    h3 = _ln(h2 + ffn, fng_ref[...], fnb_ref[...])
    h3 = h3.reshape(T1, BN, H)
    tre = tre_ref[...]
    h3 = h3 + jnp.concatenate([tre[i:i + 1] for i in range(T1)], axis=0)[:, None, :]
    hbt_ref[...] = h3
    hf2 = h3.reshape(T1 * BN, H)
    Qo_ref[...] = jnp.dot(hf2, Qw_ref[...],
                          preferred_element_type=jnp.float32).reshape(T1, BN, H)
    kk = jnp.dot(hf2, Kw_ref[...], preferred_element_type=jnp.float32).reshape(T1, BN, H)
    vv = jnp.dot(hf2, Vw_ref[...], preferred_element_type=jnp.float32).reshape(T1, BN, H)
    KVo_ref[...] = jnp.concatenate([kk, vv], axis=-1)


def _tc2(h, tre, lp):
    grid = (N // BN,)
    wspec = pl.BlockSpec((H, H), lambda i: (0, 0))
    bspec = pl.BlockSpec((H,), lambda i: (0,))
    hspec = pl.BlockSpec((T1, BN, H), lambda i: (0, i, 0))
    return pl.pallas_call(
        _tc2_body,
        grid=grid,
        in_specs=[
            hspec,
            pl.BlockSpec((8, H), lambda i: (0, 0)),
            wspec, bspec, wspec, bspec, wspec, bspec, wspec, bspec,
            bspec, bspec,
            pl.BlockSpec((H, 2 * H), lambda i: (0, 0)),
            pl.BlockSpec((2 * H,), lambda i: (0,)),
            pl.BlockSpec((2 * H, H), lambda i: (0, 0)),
            bspec,
            bspec, bspec,
            wspec, wspec, wspec,
        ],
        out_specs=[hspec, hspec, pl.BlockSpec((T1, BN, 2 * H), lambda i: (0, i, 0))],
        out_shape=[jax.ShapeDtypeStruct((T1, N, H), jnp.float32),
                   jax.ShapeDtypeStruct((T1, NROWS, H), jnp.float32),
                   jax.ShapeDtypeStruct((T1, NROWS, 2 * H), jnp.float32)],
        interpret=_INTERPRET,
    )(h, tre,
      lp['attn_wq'], lp['attn_bq'], lp['attn_wk'], lp['attn_bk'],
      lp['attn_wv'], lp['attn_bv'], lp['attn_ow'], lp['attn_ob'],
      lp['tn_g'], lp['tn_b'], lp['ffn_w1'], lp['ffn_b1'],
      lp['ffn_w2'], lp['ffn_b2'], lp['fn_g'], lp['fn_b'],
      lp['Q'], lp['K'], lp['V'])


def _tc3_body(hbt_ref, nd_ref, ow_ref, ob_ref, gng_ref, gnb_ref, ew1_ref,
              h_ref, pd_ref, ps_ref, *, first):
    b16 = _b16_64()
    nd = nd_ref[0] + nd_ref[1]
    ndf = nd.reshape(T1 * BN, 80)
    num = ndf[:, :H]
    den = jnp.dot(ndf[:, H:], b16, preferred_element_type=jnp.float32) + 1e-9
    out = num / den
    hbt = hbt_ref[...].reshape(T1 * BN, H)
    hbt = hbt + jnp.dot(out, ow_ref[...], preferred_element_type=jnp.float32) + ob_ref[...]
    if first:
        hm = hbt.reshape(T1, BN, H)
        hmean = (hm[0] + hm[1] + hm[2] + hm[3] + hm[4]) * (1.0 / T1)
        ew1 = ew1_ref[...]
        pd_ref[...] = jnp.dot(hmean, ew1[:H, :], preferred_element_type=jnp.float32)
        ps_ref[...] = jnp.dot(hmean, ew1[H:2 * H, :], preferred_element_type=jnp.float32)
    h_ref[...] = _ln(hbt, gng_ref[...], gnb_ref[...]).reshape(T1, BN, H)


def _tc3(hbt, numden, lp, first):
    grid = (N // BN,)
    hspec = pl.BlockSpec((T1, BN, H), lambda i: (0, i, 0))
    nspec = pl.BlockSpec((2, T1, BN, 80), lambda i: (0, 0, i, 0))
    pspec = pl.BlockSpec((BN, H), lambda i: (i, 0))
    out_specs = [hspec, pspec, pspec]
    out_shape = [jax.ShapeDtypeStruct((T1, N, H), jnp.float32),
                 jax.ShapeDtypeStruct((N, H), jnp.float32),
                 jax.ShapeDtypeStruct((N, H), jnp.float32)]
    res = pl.pallas_call(
        functools.partial(_tc3_body, first=first),
        grid=grid,
        in_specs=[
            hspec, nspec,
            pl.BlockSpec((H, H), lambda i: (0, 0)),
            pl.BlockSpec((H,), lambda i: (0,)),
            pl.BlockSpec((H,), lambda i: (0,)),
            pl.BlockSpec((H,), lambda i: (0,)),
            pl.BlockSpec((2 * H + DE, 4 * DE), lambda i: (0, 0)),
        ],
        out_specs=out_specs,
        out_shape=out_shape,
        interpret=_INTERPRET,
    )(hbt, numden, lp['ow'], lp['ob'], lp['gn_g'], lp['gn_b'], lp['ew1'])
    return res


def _tce0_body(gd_ref, gs_ref, wd_ref, ws_ref, epb_ref, w0_ref, b0_ref,
               e_ref, mod_ref):
    e = (jnp.dot(gd_ref[...], wd_ref[...], preferred_element_type=jnp.float32)
         + jnp.dot(gs_ref[...], ws_ref[...], preferred_element_type=jnp.float32)
         + epb_ref[...])
    e_ref[...] = e
    mod_ref[...] = jnp.dot(e, w0_ref[...], preferred_element_type=jnp.float32) + b0_ref[...]


def _tce0(gdd, gds, Wd, Ws, ep_b, w0, b0):
    grid = (E_PAD // BE,)
    return pl.pallas_call(
        _tce0_body,
        grid=grid,
        in_specs=[
            pl.BlockSpec((BE, 16), lambda i: (i, 0)),
            pl.BlockSpec((BE, 16), lambda i: (i, 0)),
            pl.BlockSpec((16, DE), lambda i: (0, 0)),
            pl.BlockSpec((16, DE), lambda i: (0, 0)),
            pl.BlockSpec((DE,), lambda i: (0,)),
            pl.BlockSpec((DE, 16), lambda i: (0, 0)),
            pl.BlockSpec((16,), lambda i: (0,)),
        ],
        out_specs=[pl.BlockSpec((BE, DE), lambda i: (i, 0)),
                   pl.BlockSpec((BE, 16), lambda i: (i, 0))],
        out_shape=[jax.ShapeDtypeStruct((E_PAD, DE), jnp.float32),
                   jax.ShapeDtypeStruct((E_PAD, 16), jnp.float32)],
        interpret=_INTERPRET,
    )(gdd, gds, Wd, Ws, ep_b, w0, b0)


def _tc4_body(e_ref, gd_ref, gs_ref, ew1_ref, eb1_ref, ew2_ref, eb2_ref,
              eng_ref, enb_ref, w1_ref, b1_ref, e_out_ref, mod_ref):
    e = e_ref[...]
    z = gd_ref[...] + gs_ref[...] + eb1_ref[...] + jnp.dot(
        e, ew1_ref[...][2 * H:, :], preferred_element_type=jnp.float32)
    em = jnp.dot(_gelu(z), ew2_ref[...], preferred_element_type=jnp.float32) + eb2_ref[...]
    e_new = _ln(e + em, eng_ref[...], enb_ref[...])
    e_out_ref[...] = e_new
    mod_ref[...] = jnp.dot(e_new, w1_ref[...], preferred_element_type=jnp.float32) + b1_ref[...]


def _tc4(e, gd, gs, lp, w1mod, b1mod):
    grid = (E_PAD // BE,)
    return pl.pallas_call(
        _tc4_body,
        grid=grid,
        in_specs=[
            pl.BlockSpec((BE, DE), lambda i: (i, 0)),
            pl.BlockSpec((BE, H), lambda i: (i, 0)),
            pl.BlockSpec((BE, H), lambda i: (i, 0)),
            pl.BlockSpec((2 * H + DE, 4 * DE), lambda i: (0, 0)),
            pl.BlockSpec((4 * DE,), lambda i: (0,)),
            pl.BlockSpec((4 * DE, DE), lambda i: (0, 0)),
            pl.BlockSpec((DE,), lambda i: (0,)),
            pl.BlockSpec((DE,), lambda i: (0,)),
            pl.BlockSpec((DE,), lambda i: (0,)),
            pl.BlockSpec((DE, 16), lambda i: (0, 0)),
            pl.BlockSpec((16,), lambda i: (0,)),
        ],
        out_specs=[pl.BlockSpec((BE, DE), lambda i: (i, 0)),
                   pl.BlockSpec((BE, 16), lambda i: (i, 0))],
        out_shape=[jax.ShapeDtypeStruct((E_PAD, DE), jnp.float32),
                   jax.ShapeDtypeStruct((E_PAD, 16), jnp.float32)],
        interpret=_INTERPRET,
    )(e, gd, gs, lp['ew1'], lp['eb1'], lp['ew2'], lp['eb2'],
      lp['en_g'], lp['en_b'], w1mod, b1mod)


def _tc5_body(h_ref, w_ref, b_ref, o_ref):
    z = jnp.dot(h_ref[...], w_ref[...], preferred_element_type=jnp.float32) + b_ref[...]
    z = z - jnp.max(z, axis=-1, keepdims=True)
    ez = jnp.exp(z)
    o_ref[...] = ez / jnp.sum(ez, axis=-1, keepdims=True)


def _tc5(h2d, out_w, out_b):
    R = h2d.shape[0]
    BR = 5000
    return pl.pallas_call(
        _tc5_body,
        grid=(R // BR,),
        in_specs=[
            pl.BlockSpec((BR, H), lambda i: (i, 0)),
            pl.BlockSpec((H, SD), lambda i: (0, 0)),
            pl.BlockSpec((SD,), lambda i: (0,)),
        ],
        out_specs=pl.BlockSpec((BR, SD), lambda i: (i, 0)),
        out_shape=jax.ShapeDtypeStruct((R, SD), jnp.float32),
        interpret=_INTERPRET,
    )(h2d, out_w, out_b)


def _tc_deg_combine_body(dp_ref, o_ref):
    d = dp_ref[0] + dp_ref[1]
    l16 = lax.broadcasted_iota(jnp.int32, (NROWS, 16), 1)
    degb = jnp.where(l16 == 0, d, 0.0)
    deg = jnp.broadcast_to(jnp.sum(degb, axis=-1, keepdims=True), (NROWS, 16))
    rows = lax.broadcasted_iota(jnp.int32, (NROWS, 16), 0)
    lanes = lax.broadcasted_iota(jnp.int32, (NROWS, 16), 1)
    degv = jnp.where((rows < N) & (lanes == 0), deg, 0.0)
    dmax = jnp.maximum(jnp.max(degv), 1.0)
    o = jnp.where(lanes == 0, deg,
                  jnp.where(lanes == 1, deg / dmax,
                            jnp.where(lanes == 2, 1.0 / jnp.maximum(deg, 1.0), 0.0)))
    o_ref[...] = o


def _tc_deg_combine(deg_partial):
    return pl.pallas_call(
        _tc_deg_combine_body,
        grid=(1,),
        in_specs=[pl.BlockSpec((2, NROWS, 16), lambda i: (0, 0, 0))],
        out_specs=pl.BlockSpec((NROWS, 16), lambda i: (0, 0)),
        out_shape=jax.ShapeDtypeStruct((NROWS, 16), jnp.float32),
        interpret=_INTERPRET,
    )(deg_partial)


def _sc_mesh():
    return plsc.VectorSubcoreMesh(core_axis_name="c", subcore_axis_name="s",
                                  num_cores=NC, num_subcores=NS)


def _zero_fill(ref, rows, width):
    z = jnp.zeros((16,), jnp.float32)

    def row(r, _):
        for u in range(width // 16):
            ref[r, pl.ds(u * 16, 16)] = z
        return ()

    lax.fori_loop(0, rows, row, ())


def _sc_deg(dstp):
    @functools.partial(
        pl.kernel,
        out_type=jax.ShapeDtypeStruct((NC, NROWS, 16), jnp.float32),
        mesh=_sc_mesh(),
        compiler_params=pltpu.CompilerParams(needs_layout_passes=False, use_tc_tiling_on_sc=False),
        scratch_types=[
            pltpu.VMEM((EC,), jnp.int32),
            pltpu.VMEM((EC, 16), jnp.float32),
            pltpu.VMEM((RT, 16), jnp.float32),
            pltpu.VMEM_SHARED((NROWS, 16), jnp.float32),
        ],
    )
    def k(dh, out, idxd, onesb, rdtmp, acc):
        cid = lax.axis_index("c")
        sid = lax.axis_index("s")
        wid = sid * NC + cid

        if True:
            ov = jnp.where(lax.iota(jnp.int32, 16) == 0, 1.0, 0.0)

            def orow(r, _):
                onesb[r, pl.ds(0, 16)] = ov
                return ()

            lax.fori_loop(0, EC, orow, ())
            _zero_fill(rdtmp, RT, 16)
            pltpu.sync_copy(rdtmp, acc.at[pl.ds(sid * RT, RT)])
            plsc.subcore_barrier()

            def chunk(j, _):
                base = wid * PT + j * EC
                pltpu.sync_copy(dh.at[pl.ds(base, EC)], idxd)
                pltpu.sync_copy(onesb, acc.at[idxd], add=True)
                return ()

            lax.fori_loop(0, NCHUNK, chunk, ())
            plsc.subcore_barrier()
            pltpu.sync_copy(acc.at[pl.ds(sid * RT, RT)], rdtmp)
            pltpu.sync_copy(rdtmp, out.at[cid, pl.ds(sid * RT, RT)])

    return k(dstp)


def _sc_gather_pair(tabD, tabS, dstp, srcp):
    W = tabD.shape[1]

    @functools.partial(
        pl.kernel,
        out_type=[jax.ShapeDtypeStruct((E_PAD, W), jnp.float32)] * 2,
        mesh=_sc_mesh(),
        compiler_params=pltpu.CompilerParams(needs_layout_passes=False, use_tc_tiling_on_sc=False),
        scratch_types=[
            pltpu.VMEM((EC,), jnp.int32),
            pltpu.VMEM((EC,), jnp.int32),
            pltpu.VMEM((EC, W), jnp.float32),
            pltpu.VMEM((EC, W), jnp.float32),
            pltpu.SemaphoreType.DMA,
        ],
    )
    def k(td, ts, dh, sh, gd, gs, idxd, idxs, bd, bs, sem):
        cid = lax.axis_index("c")
        sid = lax.axis_index("s")
        wid = sid * NC + cid

        def chunk(j, _):
            base = wid * PT + j * EC
            pltpu.sync_copy(dh.at[pl.ds(base, EC)], idxd)
            pltpu.sync_copy(sh.at[pl.ds(base, EC)], idxs)
            c1 = pltpu.async_copy(td.at[idxd], bd, sem)
            c2 = pltpu.async_copy(ts.at[idxs], bs, sem)
            c1.wait()
            c2.wait()
            c3 = pltpu.async_copy(bd, gd.at[pl.ds(base, EC)], sem)
            c4 = pltpu.async_copy(bs, gs.at[pl.ds(base, EC)], sem)
            c3.wait()
            c4.wait()
            return ()

        lax.fori_loop(0, NCHUNK, chunk, ())

    return k(tabD, tabS, dstp, srcp)


def _sc_attn(Qt, KVt, dstp, srcp, mod):
    @functools.partial(
        pl.kernel,
        out_type=jax.ShapeDtypeStruct((NC, T1, NROWS, 80), jnp.float32),
        mesh=_sc_mesh(),
        compiler_params=pltpu.CompilerParams(needs_layout_passes=False, use_tc_tiling_on_sc=False),
        scratch_types=[
            pltpu.VMEM((PT,), jnp.int32),
            pltpu.VMEM((PT,), jnp.int32),
            pltpu.VMEM((EC,), jnp.int32),
            pltpu.VMEM((EC,), jnp.int32),
            pltpu.VMEM((EC,), jnp.int32),
            pltpu.VMEM((EC,), jnp.int32),
            pltpu.VMEM((EC, H), jnp.float32),
            pltpu.VMEM((EC, 2 * H), jnp.float32),
            pltpu.VMEM((EC, 16), jnp.float32),
            pltpu.VMEM((EC, 80), jnp.float32),
            pltpu.VMEM((EC, 80), jnp.float32),
            pltpu.VMEM((RD, 80), jnp.float32),
            pltpu.SemaphoreType.DMA,
            pltpu.SemaphoreType.DMA,
            pltpu.VMEM_SHARED((NROWS, 80), jnp.float32),
        ],
    )
    def k(q_hbm, kv_hbm, dst_hbm, src_hbm, mod_hbm, out_hbm,
          idxall_d, idxall_s, idxd0, idxd1, idxq, idxk, qbuf, kvbuf, modbuf,
          contrib0, contrib1, rdtmp, sem, sem2, acc):
        cid = lax.axis_index("c")
        sid = lax.axis_index("s")
        wid = sid * NC + cid
        pltpu.sync_copy(dst_hbm.at[pl.ds(wid * PT, PT)], idxall_d)
        pltpu.sync_copy(src_hbm.at[pl.ds(wid * PT, PT)], idxall_s)

        if True:
            _zero_fill(contrib0, EC, 80)
            _zero_fill(contrib1, EC, 80)

            def t_body(t, _):
                _zero_fill(rdtmp, RD, 80)
                for u in range(NSUB):
                    pltpu.sync_copy(rdtmp, acc.at[pl.ds(sid * RT + u * RD, RD)])
                plsc.subcore_barrier()
                toff = t * NROWS
                lane = lax.iota(jnp.int32, 16)
                zero16 = jnp.zeros((16,), jnp.float32)

                def do_chunk(j, idxd, contrib):
                    base = wid * PT + j * EC

                    def mkidx(kk, _):
                        sl = pl.ds(kk * 16, 16)
                        dv = idxall_d[pl.ds(j * EC + kk * 16, 16)]
                        sv = idxall_s[pl.ds(j * EC + kk * 16, 16)]
                        idxd[sl] = dv
                        idxq[sl] = dv + toff
                        idxk[sl] = sv + toff
                        return ()

                    lax.fori_loop(0, EC // 16, mkidx, ())
                    c1 = pltpu.async_copy(q_hbm.at[idxq], qbuf, sem)
                    c2 = pltpu.async_copy(kv_hbm.at[idxk], kvbuf, sem)
                    c4 = pltpu.async_copy(mod_hbm.at[pl.ds(base, EC)], modbuf, sem)
                    c1.wait()
                    c2.wait()
                    c4.wait()

                    def edge(ee, _):
                        svec = zero16
                        mrow = modbuf[ee, pl.ds(0, 16)]
                        for h in range(NH):
                            qh = qbuf[ee, pl.ds(h * DF, DF)]
                            kh = kvbuf[ee, pl.ds(h * DF, DF)]
                            dh = jnp.sum(qh * kh)
                            sh = dh * mrow[h] + mrow[NH + h]
                            svec = svec + jnp.where(lane == h,
                                                    jnp.broadcast_to(sh, (16,)), zero16)
                        wv = jnp.exp(svec)
                        contrib[ee, pl.ds(H, 16)] = jnp.where(lane < NH, wv, zero16)
                        for h in range(NH):
                            wsc = jnp.broadcast_to(wv[h], (16,))
                            contrib[ee, pl.ds(h * DF, DF)] = kvbuf[ee, pl.ds(H + h * DF, DF)] * wsc
                        return ()

                    lax.fori_loop(0, EC, edge, ())
                    pltpu.async_copy(contrib, acc.at[idxd], sem2, add=True)
                    return ()

                def chunk2(j2, _):
                    @pl.when(j2 > 0)
                    def _():
                        pltpu.make_async_copy(contrib0, acc.at[idxd0], sem2).wait()
                    do_chunk(j2 * 2, idxd0, contrib0)

                    @pl.when(j2 > 0)
                    def _():
                        pltpu.make_async_copy(contrib1, acc.at[idxd1], sem2).wait()
                    do_chunk(j2 * 2 + 1, idxd1, contrib1)
                    return ()

                lax.fori_loop(0, NCHUNK // 2, chunk2, ())
                pltpu.make_async_copy(contrib0, acc.at[idxd0], sem2).wait()
                pltpu.make_async_copy(contrib1, acc.at[idxd1], sem2).wait()
                plsc.subcore_barrier()
                for u in range(NSUB):
                    pltpu.sync_copy(acc.at[pl.ds(sid * RT + u * RD, RD)], rdtmp)
                    pltpu.sync_copy(rdtmp, out_hbm.at[cid, t, pl.ds(sid * RT + u * RD, RD)])
                plsc.subcore_barrier()
                return ()

            lax.fori_loop(0, T1, t_body, ())

    return k(Qt, KVt, dstp, srcp, mod)


def kernel(A, X_k, k_index, edge_index, params):
    del A
    Bn = X_k.shape[0]
    loop = jnp.arange(N, dtype=edge_index.dtype)
    ei = jnp.concatenate([edge_index, jnp.stack([loop, loop])], axis=1)
    dst = ei[0].astype(jnp.int32)
    src = ei[1].astype(jnp.int32)
    npad = E_PAD - E
    dstp = jnp.concatenate([dst, jnp.full((npad,), N, jnp.int32)])
    srcp = jnp.concatenate([src, jnp.zeros((npad,), jnp.int32)])

    p = params
    temb8 = jnp.concatenate([p['time_embed'], jnp.zeros((3, TED), jnp.float32)], axis=0)
    step_row = p['step_embed'][k_index]
    Xt = X_k.reshape(N, T1, IN_SD).transpose(1, 0, 2)

    h, tre = _tc1(Xt, temb8, p['in_w'], p['in_b'], p['inn_g'], p['inn_b'],
                  step_row, p['tr_w'], p['tr_b'])

    deg_partial = _sc_deg(dstp)
    degC = _tc_deg_combine(deg_partial)
    gdd, gds = _sc_gather_pair(degC, degC, dstp, srcp)

    Wd = jnp.zeros((16, DE), jnp.float32).at[2].set(p['ep_w'][0]).at[1].set(p['ep_w'][2])
    Ws = jnp.zeros((16, DE), jnp.float32).at[1].set(p['ep_w'][1])

    def modwb(lp):
        w = jnp.concatenate([lp['emul_w'] * 0.25, lp['eadd_w'],
                             jnp.zeros((DE, 8), jnp.float32)], axis=1)
        b = jnp.concatenate([(lp['emul_b'] + 1.0) * 0.25, lp['eadd_b'],
                             jnp.zeros((8,), jnp.float32)], axis=0)
        return w, b

    w0, b0 = modwb(p['layers'][0])
    w1m, b1m = modwb(p['layers'][1])
    e, mod = _tce0(gdd, gds, Wd, Ws, p['ep_b'], w0, b0)

    for li, lp in enumerate(p['layers']):
        first = (li == 0)
        hbt, Qo, KVo = _tc2(h, tre, lp)

        ndfull = _sc_attn(Qo.reshape(T1 * NROWS, H), KVo.reshape(T1 * NROWS, 2 * H),
                          dstp, srcp, mod)
        numden = ndfull[:, :, :N, :]

        h, pd, ps = _tc3(hbt, numden, lp, first)

        if first:
            zpad = jnp.zeros((NROWS - N, H), jnp.float32)
            gd, gs = _sc_gather_pair(jnp.concatenate([pd, zpad], axis=0),
                                     jnp.concatenate([ps, zpad], axis=0),
                                     dstp, srcp)
            e, mod = _tc4(e, gd, gs, lp, w1m, b1m)

    o2d = _tc5(h.reshape(T1 * N, H), p['out_w'], p['out_b'])
    return o2d.reshape(T1, N, SD).transpose(1, 0, 2).reshape(Bn, N, T1, SD)


# speedup vs baseline: 1.3674x; 1.0046x over previous
import functools
import math
import jax, jax.numpy as jnp
from jax import lax
from jax.experimental import pallas as pl
from jax.experimental.pallas import tpu as pltpu
from jax.experimental.pallas import tpu_sc as plsc

T1 = 5
N = 10000
B = 1
H = 64
NH = 4
DF = 16
DE = 16
SD = 8
IN_SD = 24
TED = 16

EC = 128
NC, NS = 2, 16
NTILES = NC * NS
E0 = 160000
E = E0 + N
E_PAD = ((E + NTILES * EC - 1) // (NTILES * EC)) * (NTILES * EC)
PT = E_PAD // NTILES
NCHUNK = PT // EC
BN = 1000
BE = 1024
NROWS = N + 240
RT = NROWS // NS
NSUB = 4
RD = RT // NSUB

_INTERPRET = False


def _ln(x, g, b):
    m = jnp.mean(x, axis=-1, keepdims=True)
    v = jnp.var(x, axis=-1, keepdims=True)
    return (x - m) / jnp.sqrt(v + 1e-5) * g + b


def _gelu(x):
    return x * 0.5 * (1.0 + lax.erf(x * (1.0 / math.sqrt(2.0))))


def _sel64():
    r = lax.broadcasted_iota(jnp.int32, (H, H), 0) // DF
    c = lax.broadcasted_iota(jnp.int32, (H, H), 1) // DF
    return (r == c).astype(jnp.float32)


def _b16_64():
    r = lax.broadcasted_iota(jnp.int32, (16, H), 0)
    c = lax.broadcasted_iota(jnp.int32, (16, H), 1) // DF
    return (r == c).astype(jnp.float32)


def _tc1_body(x_ref, temb_ref, inw_ref, inb_ref, ing_ref, inbb_ref,
              step_ref, trw_ref, trb_ref, h_ref, tre_ref):
    temb = temb_ref[...]
    te = jnp.dot(temb, inw_ref[...][IN_SD:, :],
                 preferred_element_type=jnp.float32)
    tre_ref[...] = jnp.dot(temb, trw_ref[...],
                           preferred_element_type=jnp.float32) + trb_ref[...]
    w1 = inw_ref[...][:IN_SD, :]
    step = step_ref[...]
    for i in range(T1):
        hi = jnp.dot(x_ref[i], w1, preferred_element_type=jnp.float32)
        hi = hi + te[i:i + 1, :] + inb_ref[...]
        hi = _ln(hi, ing_ref[...], inbb_ref[...])
        h_ref[i] = hi + step


def _tc1(Xt, temb8, in_w, in_b, inn_g, inn_b, step_row, tr_w, tr_b):
    grid = (N // BN,)
    return pl.pallas_call(
        _tc1_body,
        grid=grid,
        in_specs=[
            pl.BlockSpec((T1, BN, IN_SD), lambda i: (0, i, 0)),
            pl.BlockSpec((8, TED), lambda i: (0, 0)),
            pl.BlockSpec((IN_SD + TED, H), lambda i: (0, 0)),
            pl.BlockSpec((H,), lambda i: (0,)),
            pl.BlockSpec((H,), lambda i: (0,)),
            pl.BlockSpec((H,), lambda i: (0,)),
            pl.BlockSpec((1, H), lambda i: (0, 0)),
            pl.BlockSpec((TED, H), lambda i: (0, 0)),
            pl.BlockSpec((H,), lambda i: (0,)),
        ],
        out_specs=[
            pl.BlockSpec((T1, BN, H), lambda i: (0, i, 0)),
            pl.BlockSpec((8, H), lambda i: (0, 0)),
        ],
        out_shape=[
            jax.ShapeDtypeStruct((T1, N, H), jnp.float32),
            jax.ShapeDtypeStruct((8, H), jnp.float32),
        ],
        interpret=_INTERPRET,
    )(Xt, temb8, in_w, in_b, inn_g, inn_b, step_row, tr_w, tr_b)


def _tc2_body(h_ref, tre_ref,
              wq_ref, bq_ref, wk_ref, bk_ref, wv_ref, bv_ref, ow_ref, ob_ref,
              tng_ref, tnb_ref, w1_ref, b1_ref, w2_ref, b2_ref,
              fng_ref, fnb_ref, Qw_ref, Kw_ref, Vw_ref,
              hbt_ref, Qo_ref, KVo_ref):
    sel = _sel64() * (1.0 / math.sqrt(DF))
    h = h_ref[...]
    hf = h.reshape(T1 * BN, H)
    q = jnp.dot(hf, wq_ref[...], preferred_element_type=jnp.float32) + bq_ref[...]
    k = jnp.dot(hf, wk_ref[...], preferred_element_type=jnp.float32) + bk_ref[...]
    v = jnp.dot(hf, wv_ref[...], preferred_element_type=jnp.float32) + bv_ref[...]
    q3 = q.reshape(T1, BN, H)
    k3 = k.reshape(T1, BN, H)
    v3 = v.reshape(T1, BN, H)
    ao = []
    for i in range(T1):
        s = [jnp.dot(q3[i] * k3[j], sel, preferred_element_type=jnp.float32)
             for j in range(T1)]
        m = s[0]
        for j in range(1, T1):
            m = jnp.maximum(m, s[j])
        p = [jnp.exp(s[j] - m) for j in range(T1)]
        z = p[0]
        for j in range(1, T1):
            z = z + p[j]
        acc = p[0] * v3[0]
        for j in range(1, T1):
            acc = acc + p[j] * v3[j]
        ao.append(acc / z)
    aof = jnp.stack(ao, axis=0).reshape(T1 * BN, H)
    aof = jnp.dot(aof, ow_ref[...], preferred_element_type=jnp.float32) + ob_ref[...]
    h2 = _ln(hf + aof, tng_ref[...], tnb_ref[...])
    ffn = jnp.dot(_gelu(jnp.dot(h2, w1_ref[...], preferred_element_type=jnp.float32) + b1_ref[...]),
                  w2_ref[...], preferred_element_type=jnp.float32) + b2_ref[...]
    h3 = _ln(h2 + ffn, fng_ref[...], fnb_ref[...])
    h3 = h3.reshape(T1, BN, H)
    tre = tre_ref[...]
    h3 = h3 + jnp.concatenate([tre[i:i + 1] for i in range(T1)], axis=0)[:, None, :]
    hbt_ref[...] = h3
    hf2 = h3.reshape(T1 * BN, H)
    Qo_ref[...] = jnp.dot(hf2, Qw_ref[...],
                          preferred_element_type=jnp.float32).reshape(T1, BN, H)
    kk = jnp.dot(hf2, Kw_ref[...], preferred_element_type=jnp.float32).reshape(T1, BN, H)
    vv = jnp.dot(hf2, Vw_ref[...], preferred_element_type=jnp.float32).reshape(T1, BN, H)
    KVo_ref[...] = jnp.concatenate([kk, vv], axis=-1)


def _tc2(h, tre, lp):
    grid = (N // BN,)
    wspec = pl.BlockSpec((H, H), lambda i: (0, 0))
    bspec = pl.BlockSpec((H,), lambda i: (0,))
    hspec = pl.BlockSpec((T1, BN, H), lambda i: (0, i, 0))
    return pl.pallas_call(
        _tc2_body,
        grid=grid,
        in_specs=[
            hspec,
            pl.BlockSpec((8, H), lambda i: (0, 0)),
            wspec, bspec, wspec, bspec, wspec, bspec, wspec, bspec,
            bspec, bspec,
            pl.BlockSpec((H, 2 * H), lambda i: (0, 0)),
            pl.BlockSpec((2 * H,), lambda i: (0,)),
            pl.BlockSpec((2 * H, H), lambda i: (0, 0)),
            bspec,
            bspec, bspec,
            wspec, wspec, wspec,
        ],
        out_specs=[hspec, hspec, pl.BlockSpec((T1, BN, 2 * H), lambda i: (0, i, 0))],
        out_shape=[jax.ShapeDtypeStruct((T1, N, H), jnp.float32),
                   jax.ShapeDtypeStruct((T1, NROWS, H), jnp.float32),
                   jax.ShapeDtypeStruct((T1, NROWS, 2 * H), jnp.float32)],
        interpret=_INTERPRET,
    )(h, tre,
      lp['attn_wq'], lp['attn_bq'], lp['attn_wk'], lp['attn_bk'],
      lp['attn_wv'], lp['attn_bv'], lp['attn_ow'], lp['attn_ob'],
      lp['tn_g'], lp['tn_b'], lp['ffn_w1'], lp['ffn_b1'],
      lp['ffn_w2'], lp['ffn_b2'], lp['fn_g'], lp['fn_b'],
      lp['Q'], lp['K'], lp['V'])


def _tc3_body(hbt_ref, nd_ref, ow_ref, ob_ref, gng_ref, gnb_ref, ew1_ref,
              h_ref, pd_ref, ps_ref, *, first):
    b16 = _b16_64()
    nd = nd_ref[0] + nd_ref[1]
    ndf = nd.reshape(T1 * BN, 80)
    num = ndf[:, :H]
    den = jnp.dot(ndf[:, H:], b16, preferred_element_type=jnp.float32) + 1e-9
    out = num / den
    hbt = hbt_ref[...].reshape(T1 * BN, H)
    hbt = hbt + jnp.dot(out, ow_ref[...], preferred_element_type=jnp.float32) + ob_ref[...]
    if first:
        hm = hbt.reshape(T1, BN, H)
        hmean = (hm[0] + hm[1] + hm[2] + hm[3] + hm[4]) * (1.0 / T1)
        ew1 = ew1_ref[...]
        pd_ref[...] = jnp.dot(hmean, ew1[:H, :], preferred_element_type=jnp.float32)
        ps_ref[...] = jnp.dot(hmean, ew1[H:2 * H, :], preferred_element_type=jnp.float32)
    h_ref[...] = _ln(hbt, gng_ref[...], gnb_ref[...]).reshape(T1, BN, H)


def _tc3(hbt, numden, lp, first):
    grid = (N // BN,)
    hspec = pl.BlockSpec((T1, BN, H), lambda i: (0, i, 0))
    nspec = pl.BlockSpec((2, T1, BN, 80), lambda i: (0, 0, i, 0))
    pspec = pl.BlockSpec((BN, H), lambda i: (i, 0))
    out_specs = [hspec, pspec, pspec]
    out_shape = [jax.ShapeDtypeStruct((T1, N, H), jnp.float32),
                 jax.ShapeDtypeStruct((N, H), jnp.float32),
                 jax.ShapeDtypeStruct((N, H), jnp.float32)]
    res = pl.pallas_call(
        functools.partial(_tc3_body, first=first),
        grid=grid,
        in_specs=[
            hspec, nspec,
            pl.BlockSpec((H, H), lambda i: (0, 0)),
            pl.BlockSpec((H,), lambda i: (0,)),
            pl.BlockSpec((H,), lambda i: (0,)),
            pl.BlockSpec((H,), lambda i: (0,)),
            pl.BlockSpec((2 * H + DE, 4 * DE), lambda i: (0, 0)),
        ],
        out_specs=out_specs,
        out_shape=out_shape,
        interpret=_INTERPRET,
    )(hbt, numden, lp['ow'], lp['ob'], lp['gn_g'], lp['gn_b'], lp['ew1'])
    return res


def _tce0_body(gd_ref, gs_ref, wd_ref, ws_ref, epb_ref, w0_ref, b0_ref,
               e_ref, mod_ref):
    e = (jnp.dot(gd_ref[...], wd_ref[...], preferred_element_type=jnp.float32)
         + jnp.dot(gs_ref[...], ws_ref[...], preferred_element_type=jnp.float32)
         + epb_ref[...])
    e_ref[...] = e
    mod_ref[...] = jnp.dot(e, w0_ref[...], preferred_element_type=jnp.float32) + b0_ref[...]


def _tce0(gdd, gds, Wd, Ws, ep_b, w0, b0):
    grid = (E_PAD // BE,)
    return pl.pallas_call(
        _tce0_body,
        grid=grid,
        in_specs=[
            pl.BlockSpec((BE, 16), lambda i: (i, 0)),
            pl.BlockSpec((BE, 16), lambda i: (i, 0)),
            pl.BlockSpec((16, DE), lambda i: (0, 0)),
            pl.BlockSpec((16, DE), lambda i: (0, 0)),
            pl.BlockSpec((DE,), lambda i: (0,)),
            pl.BlockSpec((DE, 16), lambda i: (0, 0)),
            pl.BlockSpec((16,), lambda i: (0,)),
        ],
        out_specs=[pl.BlockSpec((BE, DE), lambda i: (i, 0)),
                   pl.BlockSpec((BE, 16), lambda i: (i, 0))],
        out_shape=[jax.ShapeDtypeStruct((E_PAD, DE), jnp.float32),
                   jax.ShapeDtypeStruct((E_PAD, 16), jnp.float32)],
        interpret=_INTERPRET,
    )(gdd, gds, Wd, Ws, ep_b, w0, b0)


def _tc4_body(e_ref, gd_ref, gs_ref, ew1_ref, eb1_ref, ew2_ref, eb2_ref,
              eng_ref, enb_ref, w1_ref, b1_ref, e_out_ref, mod_ref):
    e = e_ref[...]
    z = gd_ref[...] + gs_ref[...] + eb1_ref[...] + jnp.dot(
        e, ew1_ref[...][2 * H:, :], preferred_element_type=jnp.float32)
    em = jnp.dot(_gelu(z), ew2_ref[...], preferred_element_type=jnp.float32) + eb2_ref[...]
    e_new = _ln(e + em, eng_ref[...], enb_ref[...])
    e_out_ref[...] = e_new
    mod_ref[...] = jnp.dot(e_new, w1_ref[...], preferred_element_type=jnp.float32) + b1_ref[...]


def _tc4(e, gd, gs, lp, w1mod, b1mod):
    grid = (E_PAD // BE,)
    return pl.pallas_call(
        _tc4_body,
        grid=grid,
        in_specs=[
            pl.BlockSpec((BE, DE), lambda i: (i, 0)),
            pl.BlockSpec((BE, H), lambda i: (i, 0)),
            pl.BlockSpec((BE, H), lambda i: (i, 0)),
            pl.BlockSpec((2 * H + DE, 4 * DE), lambda i: (0, 0)),
            pl.BlockSpec((4 * DE,), lambda i: (0,)),
            pl.BlockSpec((4 * DE, DE), lambda i: (0, 0)),
            pl.BlockSpec((DE,), lambda i: (0,)),
            pl.BlockSpec((DE,), lambda i: (0,)),
            pl.BlockSpec((DE,), lambda i: (0,)),
            pl.BlockSpec((DE, 16), lambda i: (0, 0)),
            pl.BlockSpec((16,), lambda i: (0,)),
        ],
        out_specs=[pl.BlockSpec((BE, DE), lambda i: (i, 0)),
                   pl.BlockSpec((BE, 16), lambda i: (i, 0))],
        out_shape=[jax.ShapeDtypeStruct((E_PAD, DE), jnp.float32),
                   jax.ShapeDtypeStruct((E_PAD, 16), jnp.float32)],
        interpret=_INTERPRET,
    )(e, gd, gs, lp['ew1'], lp['eb1'], lp['ew2'], lp['eb2'],
      lp['en_g'], lp['en_b'], w1mod, b1mod)


def _tc5_body(h_ref, w_ref, b_ref, o_ref):
    z = jnp.dot(h_ref[...], w_ref[...], preferred_element_type=jnp.float32) + b_ref[...]
    z = z - jnp.max(z, axis=-1, keepdims=True)
    ez = jnp.exp(z)
    o_ref[...] = ez / jnp.sum(ez, axis=-1, keepdims=True)


def _tc5(h2d, out_w, out_b):
    R = h2d.shape[0]
    BR = 5000
    return pl.pallas_call(
        _tc5_body,
        grid=(R // BR,),
        in_specs=[
            pl.BlockSpec((BR, H), lambda i: (i, 0)),
            pl.BlockSpec((H, SD), lambda i: (0, 0)),
            pl.BlockSpec((SD,), lambda i: (0,)),
        ],
        out_specs=pl.BlockSpec((BR, SD), lambda i: (i, 0)),
        out_shape=jax.ShapeDtypeStruct((R, SD), jnp.float32),
        interpret=_INTERPRET,
    )(h2d, out_w, out_b)


def _tc_deg_combine_body(dp_ref, o_ref):
    d = dp_ref[0] + dp_ref[1]
    l16 = lax.broadcasted_iota(jnp.int32, (NROWS, 16), 1)
    degb = jnp.where(l16 == 0, d, 0.0)
    deg = jnp.broadcast_to(jnp.sum(degb, axis=-1, keepdims=True), (NROWS, 16))
    rows = lax.broadcasted_iota(jnp.int32, (NROWS, 16), 0)
    lanes = lax.broadcasted_iota(jnp.int32, (NROWS, 16), 1)
    degv = jnp.where((rows < N) & (lanes == 0), deg, 0.0)
    dmax = jnp.maximum(jnp.max(degv), 1.0)
    o = jnp.where(lanes == 0, deg,
                  jnp.where(lanes == 1, deg / dmax,
                            jnp.where(lanes == 2, 1.0 / jnp.maximum(deg, 1.0), 0.0)))
    o_ref[...] = o


def _tc_deg_combine(deg_partial):
    return pl.pallas_call(
        _tc_deg_combine_body,
        grid=(1,),
        in_specs=[pl.BlockSpec((2, NROWS, 16), lambda i: (0, 0, 0))],
        out_specs=pl.BlockSpec((NROWS, 16), lambda i: (0, 0)),
        out_shape=jax.ShapeDtypeStruct((NROWS, 16), jnp.float32),
        interpret=_INTERPRET,
    )(deg_partial)


def _sc_mesh():
    return plsc.VectorSubcoreMesh(core_axis_name="c", subcore_axis_name="s",
                                  num_cores=NC, num_subcores=NS)


def _zero_fill(ref, rows, width):
    z = jnp.zeros((16,), jnp.float32)

    def row(r, _):
        for u in range(width // 16):
            ref[r, pl.ds(u * 16, 16)] = z
        return ()

    lax.fori_loop(0, rows, row, ())


def _sc_deg(dstp):
    @functools.partial(
        pl.kernel,
        out_type=jax.ShapeDtypeStruct((NC, NROWS, 16), jnp.float32),
        mesh=_sc_mesh(),
        compiler_params=pltpu.CompilerParams(needs_layout_passes=False, use_tc_tiling_on_sc=False),
        scratch_types=[
            pltpu.VMEM((EC,), jnp.int32),
            pltpu.VMEM((EC, 16), jnp.float32),
            pltpu.VMEM((RT, 16), jnp.float32),
            pltpu.VMEM_SHARED((NROWS, 16), jnp.float32),
        ],
    )
    def k(dh, out, idxd, onesb, rdtmp, acc):
        cid = lax.axis_index("c")
        sid = lax.axis_index("s")
        wid = sid * NC + cid

        if True:
            ov = jnp.where(lax.iota(jnp.int32, 16) == 0, 1.0, 0.0)

            def orow(r, _):
                onesb[r, pl.ds(0, 16)] = ov
                return ()

            lax.fori_loop(0, EC, orow, ())
            _zero_fill(rdtmp, RT, 16)
            pltpu.sync_copy(rdtmp, acc.at[pl.ds(sid * RT, RT)])
            plsc.subcore_barrier()

            def chunk(j, _):
                base = wid * PT + j * EC
                pltpu.sync_copy(dh.at[pl.ds(base, EC)], idxd)
                pltpu.sync_copy(onesb, acc.at[idxd], add=True)
                return ()

            lax.fori_loop(0, NCHUNK, chunk, ())
            plsc.subcore_barrier()
            pltpu.sync_copy(acc.at[pl.ds(sid * RT, RT)], rdtmp)
            pltpu.sync_copy(rdtmp, out.at[cid, pl.ds(sid * RT, RT)])

    return k(dstp)


def _sc_gather_pair(tabD, tabS, dstp, srcp):
    W = tabD.shape[1]

    @functools.partial(
        pl.kernel,
        out_type=[jax.ShapeDtypeStruct((E_PAD, W), jnp.float32)] * 2,
        mesh=_sc_mesh(),
        compiler_params=pltpu.CompilerParams(needs_layout_passes=False, use_tc_tiling_on_sc=False),
        scratch_types=[
            pltpu.VMEM((EC,), jnp.int32),
            pltpu.VMEM((EC,), jnp.int32),
            pltpu.VMEM((EC, W), jnp.float32),
            pltpu.VMEM((EC, W), jnp.float32),
            pltpu.SemaphoreType.DMA,
        ],
    )
    def k(td, ts, dh, sh, gd, gs, idxd, idxs, bd, bs, sem):
        cid = lax.axis_index("c")
        sid = lax.axis_index("s")
        wid = sid * NC + cid

        def chunk(j, _):
            base = wid * PT + j * EC
            pltpu.sync_copy(dh.at[pl.ds(base, EC)], idxd)
            pltpu.sync_copy(sh.at[pl.ds(base, EC)], idxs)
            c1 = pltpu.async_copy(td.at[idxd], bd, sem)
            c2 = pltpu.async_copy(ts.at[idxs], bs, sem)
            c1.wait()
            c2.wait()
            c3 = pltpu.async_copy(bd, gd.at[pl.ds(base, EC)], sem)
            c4 = pltpu.async_copy(bs, gs.at[pl.ds(base, EC)], sem)
            c3.wait()
            c4.wait()
            return ()

        lax.fori_loop(0, NCHUNK, chunk, ())

    return k(tabD, tabS, dstp, srcp)


def _sc_attn(Qt, KVt, dstp, srcp, mod):
    @functools.partial(
        pl.kernel,
        out_type=jax.ShapeDtypeStruct((NC, T1, NROWS, 80), jnp.float32),
        mesh=_sc_mesh(),
        compiler_params=pltpu.CompilerParams(needs_layout_passes=False, use_tc_tiling_on_sc=False),
        scratch_types=[
            pltpu.VMEM((PT,), jnp.int32),
            pltpu.VMEM((PT,), jnp.int32),
            pltpu.VMEM((EC,), jnp.int32),
            pltpu.VMEM((EC,), jnp.int32),
            pltpu.VMEM((EC,), jnp.int32),
            pltpu.VMEM((EC,), jnp.int32),
            pltpu.VMEM((EC, H), jnp.float32),
            pltpu.VMEM((EC, 2 * H), jnp.float32),
            pltpu.VMEM((EC, 16), jnp.float32),
            pltpu.VMEM((EC, 80), jnp.float32),
            pltpu.VMEM((EC, 80), jnp.float32),
            pltpu.VMEM((RD, 80), jnp.float32),
            pltpu.SemaphoreType.DMA,
            pltpu.SemaphoreType.DMA,
            pltpu.VMEM_SHARED((NROWS, 80), jnp.float32),
        ],
    )
    def k(q_hbm, kv_hbm, dst_hbm, src_hbm, mod_hbm, out_hbm,
          idxall_d, idxall_s, idxd0, idxd1, idxq, idxk, qbuf, kvbuf, modbuf,
          contrib0, contrib1, rdtmp, sem, sem2, acc):
        cid = lax.axis_index("c")
        sid = lax.axis_index("s")
        wid = sid * NC + cid
        pltpu.sync_copy(dst_hbm.at[pl.ds(wid * PT, PT)], idxall_d)
        pltpu.sync_copy(src_hbm.at[pl.ds(wid * PT, PT)], idxall_s)

        if True:
            _zero_fill(contrib0, EC, 80)
            _zero_fill(contrib1, EC, 80)

            def t_body(t, _):
                _zero_fill(rdtmp, RD, 80)
                for u in range(NSUB):
                    pltpu.sync_copy(rdtmp, acc.at[pl.ds(sid * RT + u * RD, RD)])
                plsc.subcore_barrier()
                toff = t * NROWS
                lane = lax.iota(jnp.int32, 16)
                zero16 = jnp.zeros((16,), jnp.float32)

                def do_chunk(j, idxd, contrib):
                    base = wid * PT + j * EC

                    def mkidx(kk, _):
                        sl = pl.ds(kk * 16, 16)
                        dv = idxall_d[pl.ds(j * EC + kk * 16, 16)]
                        sv = idxall_s[pl.ds(j * EC + kk * 16, 16)]
                        idxd[sl] = dv
                        idxq[sl] = dv + toff
                        idxk[sl] = sv + toff
                        return ()

                    lax.fori_loop(0, EC // 16, mkidx, ())
                    c1 = pltpu.async_copy(q_hbm.at[idxq], qbuf, sem)
                    c2 = pltpu.async_copy(kv_hbm.at[idxk], kvbuf, sem)
                    c4 = pltpu.async_copy(mod_hbm.at[pl.ds(base, EC)], modbuf, sem)
                    c1.wait()
                    c2.wait()
                    c4.wait()

                    def one_edge(ee):
                        svec = zero16
                        mrow = modbuf[ee, pl.ds(0, 16)]
                        for h in range(NH):
                            qh = qbuf[ee, pl.ds(h * DF, DF)]
                            kh = kvbuf[ee, pl.ds(h * DF, DF)]
                            dh = jnp.sum(qh * kh)
                            sh = dh * mrow[h] + mrow[NH + h]
                            svec = svec + jnp.where(lane == h,
                                                    jnp.broadcast_to(sh, (16,)), zero16)
                        wv = jnp.exp(svec)
                        contrib[ee, pl.ds(H, 16)] = jnp.where(lane < NH, wv, zero16)
                        for h in range(NH):
                            wsc = jnp.broadcast_to(wv[h], (16,))
                            contrib[ee, pl.ds(h * DF, DF)] = kvbuf[ee, pl.ds(H + h * DF, DF)] * wsc

                    def edge(ee, _):
                        one_edge(ee * 4)
                        one_edge(ee * 4 + 1)
                        one_edge(ee * 4 + 2)
                        one_edge(ee * 4 + 3)
                        return ()

                    lax.fori_loop(0, EC // 4, edge, ())
                    pltpu.async_copy(contrib, acc.at[idxd], sem2, add=True)
                    return ()

                def chunk2(j2, _):
                    @pl.when(j2 > 0)
                    def _():
                        pltpu.make_async_copy(contrib0, acc.at[idxd0], sem2).wait()
                    do_chunk(j2 * 2, idxd0, contrib0)

                    @pl.when(j2 > 0)
                    def _():
                        pltpu.make_async_copy(contrib1, acc.at[idxd1], sem2).wait()
                    do_chunk(j2 * 2 + 1, idxd1, contrib1)
                    return ()

                lax.fori_loop(0, NCHUNK // 2, chunk2, ())
                pltpu.make_async_copy(contrib0, acc.at[idxd0], sem2).wait()
                pltpu.make_async_copy(contrib1, acc.at[idxd1], sem2).wait()
                plsc.subcore_barrier()
                for u in range(NSUB):
                    pltpu.sync_copy(acc.at[pl.ds(sid * RT + u * RD, RD)], rdtmp)
                    pltpu.sync_copy(rdtmp, out_hbm.at[cid, t, pl.ds(sid * RT + u * RD, RD)])
                plsc.subcore_barrier()
                return ()

            lax.fori_loop(0, T1, t_body, ())

    return k(Qt, KVt, dstp, srcp, mod)


def kernel(A, X_k, k_index, edge_index, params):
    del A
    Bn = X_k.shape[0]
    loop = jnp.arange(N, dtype=edge_index.dtype)
    ei = jnp.concatenate([edge_index, jnp.stack([loop, loop])], axis=1)
    dst = ei[0].astype(jnp.int32)
    src = ei[1].astype(jnp.int32)
    npad = E_PAD - E
    dstp = jnp.concatenate([dst, jnp.full((npad,), N, jnp.int32)])
    srcp = jnp.concatenate([src, jnp.zeros((npad,), jnp.int32)])

    p = params
    temb8 = jnp.concatenate([p['time_embed'], jnp.zeros((3, TED), jnp.float32)], axis=0)
    step_row = p['step_embed'][k_index]
    Xt = X_k.reshape(N, T1, IN_SD).transpose(1, 0, 2)

    h, tre = _tc1(Xt, temb8, p['in_w'], p['in_b'], p['inn_g'], p['inn_b'],
                  step_row, p['tr_w'], p['tr_b'])

    deg_partial = _sc_deg(dstp)
    degC = _tc_deg_combine(deg_partial)
    gdd, gds = _sc_gather_pair(degC, degC, dstp, srcp)

    Wd = jnp.zeros((16, DE), jnp.float32).at[2].set(p['ep_w'][0]).at[1].set(p['ep_w'][2])
    Ws = jnp.zeros((16, DE), jnp.float32).at[1].set(p['ep_w'][1])

    def modwb(lp):
        w = jnp.concatenate([lp['emul_w'] * 0.25, lp['eadd_w'],
                             jnp.zeros((DE, 8), jnp.float32)], axis=1)
        b = jnp.concatenate([(lp['emul_b'] + 1.0) * 0.25, lp['eadd_b'],
                             jnp.zeros((8,), jnp.float32)], axis=0)
        return w, b

    w0, b0 = modwb(p['layers'][0])
    w1m, b1m = modwb(p['layers'][1])
    e, mod = _tce0(gdd, gds, Wd, Ws, p['ep_b'], w0, b0)

    for li, lp in enumerate(p['layers']):
        first = (li == 0)
        hbt, Qo, KVo = _tc2(h, tre, lp)

        ndfull = _sc_attn(Qo.reshape(T1 * NROWS, H), KVo.reshape(T1 * NROWS, 2 * H),
                          dstp, srcp, mod)
        numden = ndfull[:, :, :N, :]

        h, pd, ps = _tc3(hbt, numden, lp, first)

        if first:
            zpad = jnp.zeros((NROWS - N, H), jnp.float32)
            gd, gs = _sc_gather_pair(jnp.concatenate([pd, zpad], axis=0),
                                     jnp.concatenate([ps, zpad], axis=0),
                                     dstp, srcp)
            e, mod = _tc4(e, gd, gs, lp, w1m, b1m)

    o2d = _tc5(h.reshape(T1 * N, H), p['out_w'], p['out_b'])
    return o2d.reshape(T1, N, SD).transpose(1, 0, 2).reshape(Bn, N, T1, SD)


# speedup vs baseline: 1.3674x; 1.0000x over previous
import functools
import math
import jax, jax.numpy as jnp
from jax import lax
from jax.experimental import pallas as pl
from jax.experimental.pallas import tpu as pltpu
from jax.experimental.pallas import tpu_sc as plsc

T1 = 5
N = 10000
B = 1
H = 64
NH = 4
DF = 16
DE = 16
SD = 8
IN_SD = 24
TED = 16

EC = 128
NC, NS = 2, 16
NTILES = NC * NS
E0 = 160000
E = E0 + N
E_PAD = ((E + NTILES * EC - 1) // (NTILES * EC)) * (NTILES * EC)
PT = E_PAD // NTILES
NCHUNK = PT // EC
BN = 1000
BE = 1024
NROWS = N + 240
RT = NROWS // NS
NSUB = 4
RD = RT // NSUB


def _ln(x, g, b):
    m = jnp.mean(x, axis=-1, keepdims=True)
    v = jnp.var(x, axis=-1, keepdims=True)
    return (x - m) / jnp.sqrt(v + 1e-5) * g + b


def _gelu(x):
    return x * 0.5 * (1.0 + lax.erf(x * (1.0 / math.sqrt(2.0))))


def _sel64():
    r = lax.broadcasted_iota(jnp.int32, (H, H), 0) // DF
    c = lax.broadcasted_iota(jnp.int32, (H, H), 1) // DF
    return (r == c).astype(jnp.float32)


def _b16_64():
    r = lax.broadcasted_iota(jnp.int32, (16, H), 0)
    c = lax.broadcasted_iota(jnp.int32, (16, H), 1) // DF
    return (r == c).astype(jnp.float32)


def _tc1_body(x_ref, temb_ref, inw_ref, inb_ref, ing_ref, inbb_ref,
              step_ref, trw_ref, trb_ref, h_ref, tre_ref):
    temb = temb_ref[...]
    te = jnp.dot(temb, inw_ref[...][IN_SD:, :],
                 preferred_element_type=jnp.float32)
    tre_ref[...] = jnp.dot(temb, trw_ref[...],
                           preferred_element_type=jnp.float32) + trb_ref[...]
    w1 = inw_ref[...][:IN_SD, :]
    step = step_ref[...]
    for i in range(T1):
        hi = jnp.dot(x_ref[i], w1, preferred_element_type=jnp.float32)
        hi = hi + te[i:i + 1, :] + inb_ref[...]
        hi = _ln(hi, ing_ref[...], inbb_ref[...])
        h_ref[i] = hi + step


def _tc1(Xt, temb8, in_w, in_b, inn_g, inn_b, step_row, tr_w, tr_b):
    grid = (N // BN,)
    return pl.pallas_call(
        _tc1_body,
        grid=grid,
        in_specs=[
            pl.BlockSpec((T1, BN, IN_SD), lambda i: (0, i, 0)),
            pl.BlockSpec((8, TED), lambda i: (0, 0)),
            pl.BlockSpec((IN_SD + TED, H), lambda i: (0, 0)),
            pl.BlockSpec((H,), lambda i: (0,)),
            pl.BlockSpec((H,), lambda i: (0,)),
            pl.BlockSpec((H,), lambda i: (0,)),
            pl.BlockSpec((1, H), lambda i: (0, 0)),
            pl.BlockSpec((TED, H), lambda i: (0, 0)),
            pl.BlockSpec((H,), lambda i: (0,)),
        ],
        out_specs=[
            pl.BlockSpec((T1, BN, H), lambda i: (0, i, 0)),
            pl.BlockSpec((8, H), lambda i: (0, 0)),
        ],
        out_shape=[
            jax.ShapeDtypeStruct((T1, N, H), jnp.float32),
            jax.ShapeDtypeStruct((8, H), jnp.float32),
        ],
    )(Xt, temb8, in_w, in_b, inn_g, inn_b, step_row, tr_w, tr_b)


def _tc2_body(h_ref, tre_ref,
              wq_ref, bq_ref, wk_ref, bk_ref, wv_ref, bv_ref, ow_ref, ob_ref,
              tng_ref, tnb_ref, w1_ref, b1_ref, w2_ref, b2_ref,
              fng_ref, fnb_ref, Qw_ref, Kw_ref, Vw_ref,
              hbt_ref, Qo_ref, KVo_ref):
    sel = _sel64() * (1.0 / math.sqrt(DF))
    h = h_ref[...]
    hf = h.reshape(T1 * BN, H)
    q = jnp.dot(hf, wq_ref[...], preferred_element_type=jnp.float32) + bq_ref[...]
    k = jnp.dot(hf, wk_ref[...], preferred_element_type=jnp.float32) + bk_ref[...]
    v = jnp.dot(hf, wv_ref[...], preferred_element_type=jnp.float32) + bv_ref[...]
    q3 = q.reshape(T1, BN, H)
    k3 = k.reshape(T1, BN, H)
    v3 = v.reshape(T1, BN, H)
    ao = []
    for i in range(T1):
        s = [jnp.dot(q3[i] * k3[j], sel, preferred_element_type=jnp.float32)
             for j in range(T1)]
        m = s[0]
        for j in range(1, T1):
            m = jnp.maximum(m, s[j])
        p = [jnp.exp(s[j] - m) for j in range(T1)]
        z = p[0]
        for j in range(1, T1):
            z = z + p[j]
        acc = p[0] * v3[0]
        for j in range(1, T1):
            acc = acc + p[j] * v3[j]
        ao.append(acc / z)
    aof = jnp.stack(ao, axis=0).reshape(T1 * BN, H)
    aof = jnp.dot(aof, ow_ref[...], preferred_element_type=jnp.float32) + ob_ref[...]
    h2 = _ln(hf + aof, tng_ref[...], tnb_ref[...])
    ffn = jnp.dot(_gelu(jnp.dot(h2, w1_ref[...], preferred_element_type=jnp.float32) + b1_ref[...]),
                  w2_ref[...], preferred_element_type=jnp.float32) + b2_ref[...]
    h3 = _ln(h2 + ffn, fng_ref[...], fnb_ref[...])
    h3 = h3.reshape(T1, BN, H)
    tre = tre_ref[...]
    h3 = h3 + jnp.concatenate([tre[i:i + 1] for i in range(T1)], axis=0)[:, None, :]
    hbt_ref[...] = h3
    hf2 = h3.reshape(T1 * BN, H)
    Qo_ref[...] = jnp.dot(hf2, Qw_ref[...],
                          preferred_element_type=jnp.float32).reshape(T1, BN, H)
    kk = jnp.dot(hf2, Kw_ref[...], preferred_element_type=jnp.float32).reshape(T1, BN, H)
    vv = jnp.dot(hf2, Vw_ref[...], preferred_element_type=jnp.float32).reshape(T1, BN, H)
    KVo_ref[...] = jnp.concatenate([kk, vv], axis=-1)


def _tc2(h, tre, lp):
    grid = (N // BN,)
    wspec = pl.BlockSpec((H, H), lambda i: (0, 0))
    bspec = pl.BlockSpec((H,), lambda i: (0,))
    hspec = pl.BlockSpec((T1, BN, H), lambda i: (0, i, 0))
    return pl.pallas_call(
        _tc2_body,
        grid=grid,
        in_specs=[
            hspec,
            pl.BlockSpec((8, H), lambda i: (0, 0)),
            wspec, bspec, wspec, bspec, wspec, bspec, wspec, bspec,
            bspec, bspec,
            pl.BlockSpec((H, 2 * H), lambda i: (0, 0)),
            pl.BlockSpec((2 * H,), lambda i: (0,)),
            pl.BlockSpec((2 * H, H), lambda i: (0, 0)),
            bspec,
            bspec, bspec,
            wspec, wspec, wspec,
        ],
        out_specs=[hspec, hspec, pl.BlockSpec((T1, BN, 2 * H), lambda i: (0, i, 0))],
        out_shape=[jax.ShapeDtypeStruct((T1, N, H), jnp.float32),
                   jax.ShapeDtypeStruct((T1, NROWS, H), jnp.float32),
                   jax.ShapeDtypeStruct((T1, NROWS, 2 * H), jnp.float32)],
    )(h, tre,
      lp['attn_wq'], lp['attn_bq'], lp['attn_wk'], lp['attn_bk'],
      lp['attn_wv'], lp['attn_bv'], lp['attn_ow'], lp['attn_ob'],
      lp['tn_g'], lp['tn_b'], lp['ffn_w1'], lp['ffn_b1'],
      lp['ffn_w2'], lp['ffn_b2'], lp['fn_g'], lp['fn_b'],
      lp['Q'], lp['K'], lp['V'])


def _tc3_body(hbt_ref, nd_ref, ow_ref, ob_ref, gng_ref, gnb_ref, ew1_ref,
              h_ref, pd_ref, ps_ref, *, first):
    b16 = _b16_64()
    nd = nd_ref[0] + nd_ref[1]
    ndf = nd.reshape(T1 * BN, 80)
    num = ndf[:, :H]
    den = jnp.dot(ndf[:, H:], b16, preferred_element_type=jnp.float32) + 1e-9
    out = num / den
    hbt = hbt_ref[...].reshape(T1 * BN, H)
    hbt = hbt + jnp.dot(out, ow_ref[...], preferred_element_type=jnp.float32) + ob_ref[...]
    if first:
        hm = hbt.reshape(T1, BN, H)
        hmean = (hm[0] + hm[1] + hm[2] + hm[3] + hm[4]) * (1.0 / T1)
        ew1 = ew1_ref[...]
        pd_ref[...] = jnp.dot(hmean, ew1[:H, :], preferred_element_type=jnp.float32)
        ps_ref[...] = jnp.dot(hmean, ew1[H:2 * H, :], preferred_element_type=jnp.float32)
    h_ref[...] = _ln(hbt, gng_ref[...], gnb_ref[...]).reshape(T1, BN, H)


def _tc3(hbt, numden, lp, first):
    grid = (N // BN,)
    hspec = pl.BlockSpec((T1, BN, H), lambda i: (0, i, 0))
    nspec = pl.BlockSpec((2, T1, BN, 80), lambda i: (0, 0, i, 0))
    pspec = pl.BlockSpec((BN, H), lambda i: (i, 0))
    out_specs = [hspec, pspec, pspec]
    out_shape = [jax.ShapeDtypeStruct((T1, N, H), jnp.float32),
                 jax.ShapeDtypeStruct((N, H), jnp.float32),
                 jax.ShapeDtypeStruct((N, H), jnp.float32)]
    res = pl.pallas_call(
        functools.partial(_tc3_body, first=first),
        grid=grid,
        in_specs=[
            hspec, nspec,
            pl.BlockSpec((H, H), lambda i: (0, 0)),
            pl.BlockSpec((H,), lambda i: (0,)),
            pl.BlockSpec((H,), lambda i: (0,)),
            pl.BlockSpec((H,), lambda i: (0,)),
            pl.BlockSpec((2 * H + DE, 4 * DE), lambda i: (0, 0)),
        ],
        out_specs=out_specs,
        out_shape=out_shape,
    )(hbt, numden, lp['ow'], lp['ob'], lp['gn_g'], lp['gn_b'], lp['ew1'])
    return res


def _tce0_body(gd_ref, gs_ref, wd_ref, ws_ref, epb_ref, w0_ref, b0_ref,
               e_ref, mod_ref):
    e = (jnp.dot(gd_ref[...], wd_ref[...], preferred_element_type=jnp.float32)
         + jnp.dot(gs_ref[...], ws_ref[...], preferred_element_type=jnp.float32)
         + epb_ref[...])
    e_ref[...] = e
    mod_ref[...] = jnp.dot(e, w0_ref[...], preferred_element_type=jnp.float32) + b0_ref[...]


def _tce0(gdd, gds, Wd, Ws, ep_b, w0, b0):
    grid = (E_PAD // BE,)
    return pl.pallas_call(
        _tce0_body,
        grid=grid,
        in_specs=[
            pl.BlockSpec((BE, 16), lambda i: (i, 0)),
            pl.BlockSpec((BE, 16), lambda i: (i, 0)),
            pl.BlockSpec((16, DE), lambda i: (0, 0)),
            pl.BlockSpec((16, DE), lambda i: (0, 0)),
            pl.BlockSpec((DE,), lambda i: (0,)),
            pl.BlockSpec((DE, 16), lambda i: (0, 0)),
            pl.BlockSpec((16,), lambda i: (0,)),
        ],
        out_specs=[pl.BlockSpec((BE, DE), lambda i: (i, 0)),
                   pl.BlockSpec((BE, 16), lambda i: (i, 0))],
        out_shape=[jax.ShapeDtypeStruct((E_PAD, DE), jnp.float32),
                   jax.ShapeDtypeStruct((E_PAD, 16), jnp.float32)],
    )(gdd, gds, Wd, Ws, ep_b, w0, b0)


def _tc4_body(e_ref, gd_ref, gs_ref, ew1_ref, eb1_ref, ew2_ref, eb2_ref,
              eng_ref, enb_ref, w1_ref, b1_ref, e_out_ref, mod_ref):
    e = e_ref[...]
    z = gd_ref[...] + gs_ref[...] + eb1_ref[...] + jnp.dot(
        e, ew1_ref[...][2 * H:, :], preferred_element_type=jnp.float32)
    em = jnp.dot(_gelu(z), ew2_ref[...], preferred_element_type=jnp.float32) + eb2_ref[...]
    e_new = _ln(e + em, eng_ref[...], enb_ref[...])
    e_out_ref[...] = e_new
    mod_ref[...] = jnp.dot(e_new, w1_ref[...], preferred_element_type=jnp.float32) + b1_ref[...]


def _tc4(e, gd, gs, lp, w1mod, b1mod):
    grid = (E_PAD // BE,)
    return pl.pallas_call(
        _tc4_body,
        grid=grid,
        in_specs=[
            pl.BlockSpec((BE, DE), lambda i: (i, 0)),
            pl.BlockSpec((BE, H), lambda i: (i, 0)),
            pl.BlockSpec((BE, H), lambda i: (i, 0)),
            pl.BlockSpec((2 * H + DE, 4 * DE), lambda i: (0, 0)),
            pl.BlockSpec((4 * DE,), lambda i: (0,)),
            pl.BlockSpec((4 * DE, DE), lambda i: (0, 0)),
            pl.BlockSpec((DE,), lambda i: (0,)),
            pl.BlockSpec((DE,), lambda i: (0,)),
            pl.BlockSpec((DE,), lambda i: (0,)),
            pl.BlockSpec((DE, 16), lambda i: (0, 0)),
            pl.BlockSpec((16,), lambda i: (0,)),
        ],
        out_specs=[pl.BlockSpec((BE, DE), lambda i: (i, 0)),
                   pl.BlockSpec((BE, 16), lambda i: (i, 0))],
        out_shape=[jax.ShapeDtypeStruct((E_PAD, DE), jnp.float32),
                   jax.ShapeDtypeStruct((E_PAD, 16), jnp.float32)],
    )(e, gd, gs, lp['ew1'], lp['eb1'], lp['ew2'], lp['eb2'],
      lp['en_g'], lp['en_b'], w1mod, b1mod)


def _tc5_body(h_ref, w_ref, b_ref, o_ref):
    z = jnp.dot(h_ref[...], w_ref[...], preferred_element_type=jnp.float32) + b_ref[...]
    z = z - jnp.max(z, axis=-1, keepdims=True)
    ez = jnp.exp(z)
    o_ref[...] = ez / jnp.sum(ez, axis=-1, keepdims=True)


def _tc5(h2d, out_w, out_b):
    R = h2d.shape[0]
    BR = 5000
    return pl.pallas_call(
        _tc5_body,
        grid=(R // BR,),
        in_specs=[
            pl.BlockSpec((BR, H), lambda i: (i, 0)),
            pl.BlockSpec((H, SD), lambda i: (0, 0)),
            pl.BlockSpec((SD,), lambda i: (0,)),
        ],
        out_specs=pl.BlockSpec((BR, SD), lambda i: (i, 0)),
        out_shape=jax.ShapeDtypeStruct((R, SD), jnp.float32),
    )(h2d, out_w, out_b)


def _tc_deg_combine_body(dp_ref, o_ref):
    d = dp_ref[0] + dp_ref[1]
    l16 = lax.broadcasted_iota(jnp.int32, (NROWS, 16), 1)
    degb = jnp.where(l16 == 0, d, 0.0)
    deg = jnp.broadcast_to(jnp.sum(degb, axis=-1, keepdims=True), (NROWS, 16))
    rows = lax.broadcasted_iota(jnp.int32, (NROWS, 16), 0)
    lanes = lax.broadcasted_iota(jnp.int32, (NROWS, 16), 1)
    degv = jnp.where((rows < N) & (lanes == 0), deg, 0.0)
    dmax = jnp.maximum(jnp.max(degv), 1.0)
    o = jnp.where(lanes == 0, deg,
                  jnp.where(lanes == 1, deg / dmax,
                            jnp.where(lanes == 2, 1.0 / jnp.maximum(deg, 1.0), 0.0)))
    o_ref[...] = o


def _tc_deg_combine(deg_partial):
    return pl.pallas_call(
        _tc_deg_combine_body,
        grid=(1,),
        in_specs=[pl.BlockSpec((2, NROWS, 16), lambda i: (0, 0, 0))],
        out_specs=pl.BlockSpec((NROWS, 16), lambda i: (0, 0)),
        out_shape=jax.ShapeDtypeStruct((NROWS, 16), jnp.float32),
    )(deg_partial)


def _sc_mesh():
    return plsc.VectorSubcoreMesh(core_axis_name="c", subcore_axis_name="s",
                                  num_cores=NC, num_subcores=NS)


def _zero_fill(ref, rows, width):
    z = jnp.zeros((16,), jnp.float32)

    def row(r, _):
        for u in range(width // 16):
            ref[r, pl.ds(u * 16, 16)] = z
        return ()

    lax.fori_loop(0, rows, row, ())


def _sc_deg(dstp):
    @functools.partial(
        pl.kernel,
        out_type=jax.ShapeDtypeStruct((NC, NROWS, 16), jnp.float32),
        mesh=_sc_mesh(),
        compiler_params=pltpu.CompilerParams(needs_layout_passes=False, use_tc_tiling_on_sc=False),
        scratch_types=[
            pltpu.VMEM((EC,), jnp.int32),
            pltpu.VMEM((EC, 16), jnp.float32),
            pltpu.VMEM((RT, 16), jnp.float32),
            pltpu.VMEM_SHARED((NROWS, 16), jnp.float32),
        ],
    )
    def k(dh, out, idxd, onesb, rdtmp, acc):
        cid = lax.axis_index("c")
        sid = lax.axis_index("s")
        wid = sid * NC + cid

        if True:
            ov = jnp.where(lax.iota(jnp.int32, 16) == 0, 1.0, 0.0)

            def orow(r, _):
                onesb[r, pl.ds(0, 16)] = ov
                return ()

            lax.fori_loop(0, EC, orow, ())
            _zero_fill(rdtmp, RT, 16)
            pltpu.sync_copy(rdtmp, acc.at[pl.ds(sid * RT, RT)])
            plsc.subcore_barrier()

            def chunk(j, _):
                base = wid * PT + j * EC
                pltpu.sync_copy(dh.at[pl.ds(base, EC)], idxd)
                pltpu.sync_copy(onesb, acc.at[idxd], add=True)
                return ()

            lax.fori_loop(0, NCHUNK, chunk, ())
            plsc.subcore_barrier()
            pltpu.sync_copy(acc.at[pl.ds(sid * RT, RT)], rdtmp)
            pltpu.sync_copy(rdtmp, out.at[cid, pl.ds(sid * RT, RT)])

    return k(dstp)


def _sc_gather_pair(tabD, tabS, dstp, srcp):
    W = tabD.shape[1]

    @functools.partial(
        pl.kernel,
        out_type=[jax.ShapeDtypeStruct((E_PAD, W), jnp.float32)] * 2,
        mesh=_sc_mesh(),
        compiler_params=pltpu.CompilerParams(needs_layout_passes=False, use_tc_tiling_on_sc=False),
        scratch_types=[
            pltpu.VMEM((EC,), jnp.int32),
            pltpu.VMEM((EC,), jnp.int32),
            pltpu.VMEM((EC, W), jnp.float32),
            pltpu.VMEM((EC, W), jnp.float32),
            pltpu.SemaphoreType.DMA,
        ],
    )
    def k(td, ts, dh, sh, gd, gs, idxd, idxs, bd, bs, sem):
        cid = lax.axis_index("c")
        sid = lax.axis_index("s")
        wid = sid * NC + cid

        def chunk(j, _):
            base = wid * PT + j * EC
            pltpu.sync_copy(dh.at[pl.ds(base, EC)], idxd)
            pltpu.sync_copy(sh.at[pl.ds(base, EC)], idxs)
            c1 = pltpu.async_copy(td.at[idxd], bd, sem)
            c2 = pltpu.async_copy(ts.at[idxs], bs, sem)
            c1.wait()
            c2.wait()
            c3 = pltpu.async_copy(bd, gd.at[pl.ds(base, EC)], sem)
            c4 = pltpu.async_copy(bs, gs.at[pl.ds(base, EC)], sem)
            c3.wait()
            c4.wait()
            return ()

        lax.fori_loop(0, NCHUNK, chunk, ())

    return k(tabD, tabS, dstp, srcp)


def _sc_attn(Qt, KVt, dstp, srcp, mod):
    @functools.partial(
        pl.kernel,
        out_type=jax.ShapeDtypeStruct((NC, T1, NROWS, 80), jnp.float32),
        mesh=_sc_mesh(),
        compiler_params=pltpu.CompilerParams(needs_layout_passes=False, use_tc_tiling_on_sc=False),
        scratch_types=[
            pltpu.VMEM((PT,), jnp.int32),
            pltpu.VMEM((PT,), jnp.int32),
            pltpu.VMEM((EC,), jnp.int32),
            pltpu.VMEM((EC,), jnp.int32),
            pltpu.VMEM((EC,), jnp.int32),
            pltpu.VMEM((EC,), jnp.int32),
            pltpu.VMEM((EC, H), jnp.float32),
            pltpu.VMEM((EC, 2 * H), jnp.float32),
            pltpu.VMEM((EC, 16), jnp.float32),
            pltpu.VMEM((EC, 80), jnp.float32),
            pltpu.VMEM((EC, 80), jnp.float32),
            pltpu.VMEM((RD, 80), jnp.float32),
            pltpu.SemaphoreType.DMA,
            pltpu.SemaphoreType.DMA,
            pltpu.VMEM_SHARED((NROWS, 80), jnp.float32),
        ],
    )
    def k(q_hbm, kv_hbm, dst_hbm, src_hbm, mod_hbm, out_hbm,
          idxall_d, idxall_s, idxd0, idxd1, idxq, idxk, qbuf, kvbuf, modbuf,
          contrib0, contrib1, rdtmp, sem, sem2, acc):
        cid = lax.axis_index("c")
        sid = lax.axis_index("s")
        wid = sid * NC + cid
        pltpu.sync_copy(dst_hbm.at[pl.ds(wid * PT, PT)], idxall_d)
        pltpu.sync_copy(src_hbm.at[pl.ds(wid * PT, PT)], idxall_s)

        if True:
            _zero_fill(contrib0, EC, 80)
            _zero_fill(contrib1, EC, 80)

            def t_body(t, _):
                _zero_fill(rdtmp, RD, 80)
                for u in range(NSUB):
                    pltpu.sync_copy(rdtmp, acc.at[pl.ds(sid * RT + u * RD, RD)])
                plsc.subcore_barrier()
                toff = t * NROWS
                lane = lax.iota(jnp.int32, 16)
                zero16 = jnp.zeros((16,), jnp.float32)

                def do_chunk(j, idxd, contrib):
                    base = wid * PT + j * EC

                    def mkidx(kk, _):
                        sl = pl.ds(kk * 16, 16)
                        dv = idxall_d[pl.ds(j * EC + kk * 16, 16)]
                        sv = idxall_s[pl.ds(j * EC + kk * 16, 16)]
                        idxd[sl] = dv
                        idxq[sl] = dv + toff
                        idxk[sl] = sv + toff
                        return ()

                    lax.fori_loop(0, EC // 16, mkidx, ())
                    c1 = pltpu.async_copy(q_hbm.at[idxq], qbuf, sem)
                    c2 = pltpu.async_copy(kv_hbm.at[idxk], kvbuf, sem)
                    c4 = pltpu.async_copy(mod_hbm.at[pl.ds(base, EC)], modbuf, sem)
                    c1.wait()
                    c2.wait()
                    c4.wait()

                    def one_edge(ee):
                        svec = zero16
                        mrow = modbuf[ee, pl.ds(0, 16)]
                        for h in range(NH):
                            qh = qbuf[ee, pl.ds(h * DF, DF)]
                            kh = kvbuf[ee, pl.ds(h * DF, DF)]
                            dh = jnp.sum(qh * kh)
                            sh = dh * mrow[h] + mrow[NH + h]
                            svec = svec + jnp.where(lane == h,
                                                    jnp.broadcast_to(sh, (16,)), zero16)
                        wv = jnp.exp(svec)
                        contrib[ee, pl.ds(H, 16)] = jnp.where(lane < NH, wv, zero16)
                        for h in range(NH):
                            wsc = jnp.broadcast_to(wv[h], (16,))
                            contrib[ee, pl.ds(h * DF, DF)] = kvbuf[ee, pl.ds(H + h * DF, DF)] * wsc

                    def edge(ee, _):
                        one_edge(ee * 4)
                        one_edge(ee * 4 + 1)
                        one_edge(ee * 4 + 2)
                        one_edge(ee * 4 + 3)
                        return ()

                    lax.fori_loop(0, EC // 4, edge, ())
                    pltpu.async_copy(contrib, acc.at[idxd], sem2, add=True)
                    return ()

                def chunk2(j2, _):
                    @pl.when(j2 > 0)
                    def _():
                        pltpu.make_async_copy(contrib0, acc.at[idxd0], sem2).wait()
                    do_chunk(j2 * 2, idxd0, contrib0)

                    @pl.when(j2 > 0)
                    def _():
                        pltpu.make_async_copy(contrib1, acc.at[idxd1], sem2).wait()
                    do_chunk(j2 * 2 + 1, idxd1, contrib1)
                    return ()

                lax.fori_loop(0, NCHUNK // 2, chunk2, ())
                pltpu.make_async_copy(contrib0, acc.at[idxd0], sem2).wait()
                pltpu.make_async_copy(contrib1, acc.at[idxd1], sem2).wait()
                plsc.subcore_barrier()
                for u in range(NSUB):
                    pltpu.sync_copy(acc.at[pl.ds(sid * RT + u * RD, RD)], rdtmp)
                    pltpu.sync_copy(rdtmp, out_hbm.at[cid, t, pl.ds(sid * RT + u * RD, RD)])
                plsc.subcore_barrier()
                return ()

            lax.fori_loop(0, T1, t_body, ())

    return k(Qt, KVt, dstp, srcp, mod)


def kernel(A, X_k, k_index, edge_index, params):
    del A
    Bn = X_k.shape[0]
    loop = jnp.arange(N, dtype=edge_index.dtype)
    ei = jnp.concatenate([edge_index, jnp.stack([loop, loop])], axis=1)
    dst = ei[0].astype(jnp.int32)
    src = ei[1].astype(jnp.int32)
    npad = E_PAD - E
    dstp = jnp.concatenate([dst, jnp.full((npad,), N, jnp.int32)])
    srcp = jnp.concatenate([src, jnp.zeros((npad,), jnp.int32)])

    p = params
    temb8 = jnp.concatenate([p['time_embed'], jnp.zeros((3, TED), jnp.float32)], axis=0)
    step_row = p['step_embed'][k_index]
    Xt = X_k.reshape(N, T1, IN_SD).transpose(1, 0, 2)

    h, tre = _tc1(Xt, temb8, p['in_w'], p['in_b'], p['inn_g'], p['inn_b'],
                  step_row, p['tr_w'], p['tr_b'])

    deg_partial = _sc_deg(dstp)
    degC = _tc_deg_combine(deg_partial)
    gdd, gds = _sc_gather_pair(degC, degC, dstp, srcp)

    Wd = jnp.zeros((16, DE), jnp.float32).at[2].set(p['ep_w'][0]).at[1].set(p['ep_w'][2])
    Ws = jnp.zeros((16, DE), jnp.float32).at[1].set(p['ep_w'][1])

    def modwb(lp):
        w = jnp.concatenate([lp['emul_w'] * 0.25, lp['eadd_w'],
                             jnp.zeros((DE, 8), jnp.float32)], axis=1)
        b = jnp.concatenate([(lp['emul_b'] + 1.0) * 0.25, lp['eadd_b'],
                             jnp.zeros((8,), jnp.float32)], axis=0)
        return w, b

    w0, b0 = modwb(p['layers'][0])
    w1m, b1m = modwb(p['layers'][1])
    e, mod = _tce0(gdd, gds, Wd, Ws, p['ep_b'], w0, b0)

    for li, lp in enumerate(p['layers']):
        first = (li == 0)
        hbt, Qo, KVo = _tc2(h, tre, lp)

        ndfull = _sc_attn(Qo.reshape(T1 * NROWS, H), KVo.reshape(T1 * NROWS, 2 * H),
                          dstp, srcp, mod)
        numden = ndfull[:, :, :N, :]

        h, pd, ps = _tc3(hbt, numden, lp, first)

        if first:
            zpad = jnp.zeros((NROWS - N, H), jnp.float32)
            gd, gs = _sc_gather_pair(jnp.concatenate([pd, zpad], axis=0),
                                     jnp.concatenate([ps, zpad], axis=0),
                                     dstp, srcp)
            e, mod = _tc4(e, gd, gs, lp, w1m, b1m)

    o2d = _tc5(h.reshape(T1 * N, H), p['out_w'], p['out_b'])
    return o2d.reshape(T1, N, SD).transpose(1, 0, 2).reshape(Bn, N, T1, SD)
